# baseline (speedup 1.0000x reference)
import functools

import jax
import jax.numpy as jnp
from jax import lax
from jax.experimental import pallas as pl
from jax.experimental.pallas import tpu as pltpu
from jax.experimental.pallas import tpu_sc as plsc

ALPHA_C = 0.5
NBLK = 1024
EBLK = 128
NW = 32
LSM_BLK = 4000


def _ceil_to(x, m):
    return (x + m - 1) // m * m


def _prep_body(x_ref, deg_ref, xs_ref, xo_ref, inv_ref):
    deg = deg_ref[...].astype(jnp.float32)
    inv = jnp.where(deg > 0, lax.rsqrt(jnp.maximum(deg, 1.0)), 0.0)
    x = x_ref[...]
    xo_ref[...] = inv[:, 0:1] * x
    xs_ref[...] = inv[:, 1:2] * x
    inv_ref[...] = inv


def _prep(xp, deg2):
    npad = xp.shape[0]
    grid = (npad // NBLK,)
    return pl.pallas_call(
        _prep_body,
        grid=grid,
        in_specs=[
            pl.BlockSpec((NBLK, 128), lambda i: (i, 0)),
            pl.BlockSpec((NBLK, 2), lambda i: (i, 0)),
        ],
        out_specs=[
            pl.BlockSpec((NBLK, 128), lambda i: (i, 0)),
            pl.BlockSpec((NBLK, 128), lambda i: (i, 0)),
            pl.BlockSpec((NBLK, 2), lambda i: (i, 0)),
        ],
        out_shape=[
            jax.ShapeDtypeStruct((npad, 128), jnp.float32),
            jax.ShapeDtypeStruct((npad, 128), jnp.float32),
            jax.ShapeDtypeStruct((npad, 2), jnp.float32),
        ],
    )(xp, deg2)


def _layer_out(x, sout, sin, inv, deg, m4, otab, itab, wof, wif,
               wsd, wds, wfc, b3):
    out_inv = inv[:, 0:1]
    in_inv = inv[:, 1:2]
    out_nei = out_inv * sout
    in_nei = in_inv * sin
    nb = x.shape[0]
    to = otab.shape[0]
    ti = itab.shape[0]
    oh_o = (lax.broadcasted_iota(jnp.int32, (nb, to), 1) == deg[:, 0:1]
            ).astype(jnp.float32)
    oh_i = (lax.broadcasted_iota(jnp.int32, (nb, ti), 1) == deg[:, 1:2]
            ).astype(jnp.float32)
    tgo = jnp.dot(oh_o, otab, preferred_element_type=jnp.float32)
    tgi = jnp.dot(oh_i, itab, preferred_element_type=jnp.float32)
    co = jnp.dot(out_nei - x + tgo, wof, preferred_element_type=jnp.float32)
    co = co + b3[3:4, 0:1]
    ci = jnp.dot(in_nei - x + tgi, wif, preferred_element_type=jnp.float32)
    ci = ci + b3[3:4, 1:2]
    c0 = 1.0 / (1.0 + jnp.exp(ci - co))
    c1 = 1.0 - c0
    c_out = c0 * m4[:, 0:1] + m4[:, 1:2]
    c_in = c1 * m4[:, 2:3] + m4[:, 3:4]
    h = (c_out * (jnp.dot(out_nei, wsd, preferred_element_type=jnp.float32)
                  + b3[0:1, :])
         + c_in * (jnp.dot(in_nei, wds, preferred_element_type=jnp.float32)
                   + b3[1:2, :])
         + ALPHA_C * (jnp.dot(x, wfc, preferred_element_type=jnp.float32)
                      + b3[2:3, :]))
    xn = jnp.maximum(h, 0.0)
    return xn, c_out, c_in


def _b0_body(x_ref, po0, po1, pi0, pi1, inv_ref, deg_ref, m4_ref,
             otab_ref, itab_ref, wof_ref, wif_ref, wsd_ref, wds_ref,
             wfc_ref, b3_ref, x1_ref, xs1_ref, xo1_ref, cp_ref):
    sout = po0[0] + po1[0]
    sin = pi0[0] + pi1[0]
    xn, c_out, c_in = _layer_out(
        x_ref[...], sout, sin, inv_ref[...], deg_ref[...], m4_ref[...],
        otab_ref[...], itab_ref[...], wof_ref[...], wif_ref[...],
        wsd_ref[...], wds_ref[...], wfc_ref[...], b3_ref[...])
    x1_ref[...] = xn
    inv = inv_ref[...]
    xo1_ref[...] = inv[:, 0:1] * xn
    xs1_ref[...] = inv[:, 1:2] * xn
    cp_ref[...] = jnp.concatenate([c_out, c_in], axis=1)


def _b1_body(x_ref, po0, po1, pi0, pi1, inv_ref, deg_ref, m4_ref,
             otab_ref, itab_ref, wof_ref, wif_ref, wsd_ref, wds_ref,
             wfc_ref, b3_ref, cp0_ref, w1_ref, w2_ref, blin_ref,
             p1_ref, p2_ref, cpf_ref):
    sout = po0[0] + po1[0]
    sin = pi0[0] + pi1[0]
    x1 = x_ref[...]
    xn, c_out, c_in = _layer_out(
        x1, sout, sin, inv_ref[...], deg_ref[...], m4_ref[...],
        otab_ref[...], itab_ref[...], wof_ref[...], wif_ref[...],
        wsd_ref[...], wds_ref[...], wfc_ref[...], b3_ref[...])
    xj = jnp.maximum(x1, xn)
    nb = xj.shape[0]
    p1 = jnp.dot(xj, w1_ref[...], preferred_element_type=jnp.float32)
    p1 = p1 + blin_ref[0:1, :]
    p2 = jnp.dot(xj, w2_ref[...], preferred_element_type=jnp.float32)
    p1_ref[...] = jnp.concatenate(
        [p1, jnp.full((nb, 8), -1e30, dtype=jnp.float32)], axis=1)
    p2_ref[...] = jnp.concatenate(
        [p2, jnp.zeros((nb, 8), dtype=jnp.float32)], axis=1)
    cpf_ref[...] = (cp0_ref[...] + jnp.concatenate([c_out, c_in], axis=1)) * 0.5


def _dense_layer(lidx, xp, part_out, part_in, inv2, deg2, m4,
                 otab, itab, wof, wif, wsd, wds, wfc, b3,
                 cp0=None, w1=None, w2=None, blin=None):
    npad = xp.shape[0]
    grid = (npad // NBLK,)
    to = otab.shape[0]
    ti = itab.shape[0]
    node_spec = lambda w: pl.BlockSpec((NBLK, w), lambda i: (i, 0))
    part_spec = lambda c: pl.BlockSpec((1, NBLK, 128), lambda i, c=c: (c, i, 0))
    full = lambda shape: pl.BlockSpec(shape, lambda i, s=shape: tuple(
        0 for _ in s))
    in_specs = [
        node_spec(128),
        part_spec(0), part_spec(1),
        part_spec(0), part_spec(1),
        node_spec(2), node_spec(2), node_spec(4),
        full((to, 128)), full((ti, 128)),
        full((128, 1)), full((128, 1)),
        full((128, 128)), full((128, 128)), full((128, 128)),
        full((8, 128)),
    ]
    args = [xp, part_out, part_out, part_in, part_in, inv2, deg2, m4,
            otab, itab, wof, wif, wsd, wds, wfc, b3]
    if lidx == 0:
        out_specs = [node_spec(128), node_spec(128), node_spec(128),
                     node_spec(2)]
        out_shape = [jax.ShapeDtypeStruct((npad, 128), jnp.float32)] * 3 + [
            jax.ShapeDtypeStruct((npad, 2), jnp.float32)]
        body = _b0_body
    else:
        in_specs += [node_spec(2), full((128, 40)), full((128, 40)),
                     full((8, 40))]
        args += [cp0, w1, w2, blin]
        out_specs = [node_spec(48), node_spec(48), node_spec(2)]
        out_shape = [jax.ShapeDtypeStruct((npad, 48), jnp.float32)] * 2 + [
            jax.ShapeDtypeStruct((npad, 2), jnp.float32)]
        body = _b1_body
    return pl.pallas_call(
        body, grid=grid, in_specs=in_specs, out_specs=out_specs,
        out_shape=out_shape)(*args)


def _lsm_body(z_ref, o_ref):
    z = z_ref[...]
    m = jnp.max(z, axis=1, keepdims=True)
    e = jnp.exp(z - m)
    s = jnp.sum(e, axis=1, keepdims=True)
    o_ref[...] = z[:, :40] - (m + jnp.log(s))


def _log_softmax(z, e_out):
    grid = (e_out // LSM_BLK,)
    return pl.pallas_call(
        _lsm_body,
        grid=grid,
        in_specs=[pl.BlockSpec((LSM_BLK, 48), lambda i: (i, 0))],
        out_specs=pl.BlockSpec((LSM_BLK, 40), lambda i: (i, 0)),
        out_shape=jax.ShapeDtypeStruct((e_out, 40), jnp.float32),
    )(z)


_NBUF = 2


@functools.lru_cache(maxsize=None)
def _make_conv_kernel(npad, nbkp):
    pw = nbkp // NW
    hw = pw // 2
    assert pw % 2 == 0 and hw % 8 == 0 and pw % _NBUF == 0
    rows_z = npad // 16
    mesh = plsc.VectorSubcoreMesh(core_axis_name="c", subcore_axis_name="s")

    @functools.partial(
        pl.kernel,
        mesh=mesh,
        name="conv_segsum",
        out_type=(
            jax.ShapeDtypeStruct((2, npad, 128), jnp.float32),
            jax.ShapeDtypeStruct((2, npad, 128), jnp.float32),
        ),
        scratch_types=(
            [pltpu.VMEM((pw, EBLK), jnp.int32),
             pltpu.VMEM((hw, EBLK), jnp.int32)]
            + [pltpu.VMEM((EBLK, 128), jnp.float32) for _ in range(_NBUF)]
            + [pltpu.VMEM_SHARED((npad, 128), jnp.float32)]
            + [pltpu.SemaphoreType.DMA for _ in range(_NBUF)]
        ),
    )
    def conv(xs_hbm, xo_hbm, row2d, col2d, zeros_hbm, sout_hbm, sin_hbm,
             gbuf, sbuf, d0, d1, acc, s0, s1):
        dbufs = (d0, d1)
        sems = (s0, s1)
        c = lax.axis_index("c")
        s = lax.axis_index("s")
        w = s * 2 + c
        start = w * pw

        def phase(table_hbm, g2d, s2d, out_prev, out_hbm):
            pltpu.sync_copy(g2d.at[pl.ds(start, pw)], gbuf)
            pltpu.sync_copy(s2d.at[pl.ds(start, hw)], sbuf)
            for b in range(_NBUF):
                pltpu.async_copy(table_hbm.at[gbuf.at[b]], dbufs[b], sems[b])
            if out_prev is not None:
                pltpu.sync_copy(acc.at[pl.ds(s * rows_z, rows_z)],
                                out_prev.at[c, pl.ds(s * rows_z, rows_z)])
            pltpu.sync_copy(zeros_hbm.at[pl.ds(s * rows_z, rows_z)],
                            acc.at[pl.ds(s * rows_z, rows_z)])
            plsc.subcore_barrier()

            def body(i, carry):
                for b in range(_NBUF):
                    j = i * _NBUF + b

                    @pl.when(j == hw)
                    def _():
                        pltpu.sync_copy(s2d.at[pl.ds(start + hw, hw)], sbuf)

                    jl = lax.select(j >= hw, j - hw, j)
                    pltpu.make_async_copy(table_hbm.at[gbuf.at[j]], dbufs[b],
                                          sems[b]).wait()
                    pltpu.sync_copy(dbufs[b], acc.at[sbuf.at[jl]], add=True)
                    jn = j + _NBUF

                    @pl.when(jn < pw)
                    def _():
                        pltpu.async_copy(table_hbm.at[gbuf.at[jn]], dbufs[b],
                                         sems[b])
                return carry

            lax.fori_loop(0, pw // _NBUF, body, 0)
            plsc.subcore_barrier()

        phase(xs_hbm, col2d, row2d, None, sout_hbm)
        phase(xo_hbm, row2d, col2d, sout_hbm, sin_hbm)
        pltpu.sync_copy(acc.at[pl.ds(s * rows_z, rows_z)],
                        sin_hbm.at[c, pl.ds(s * rows_z, rows_z)])

    return conv


_PNBUF = 4


@functools.lru_cache(maxsize=None)
def _make_pair_kernel(npad, nbkp):
    pw = nbkp // NW
    assert pw % _PNBUF == 0
    ep = nbkp * EBLK
    mesh = plsc.VectorSubcoreMesh(core_axis_name="c", subcore_axis_name="s")

    @functools.partial(
        pl.kernel,
        mesh=mesh,
        out_type=jax.ShapeDtypeStruct((ep, 48), jnp.float32),
        compiler_params=pltpu.CompilerParams(use_tc_tiling_on_sc=False,
                                             needs_layout_passes=False),
        scratch_types=(
            [pltpu.VMEM((pw, EBLK), jnp.int32),
             pltpu.VMEM((pw, EBLK), jnp.int32)]
            + [pltpu.VMEM((EBLK, 48), jnp.float32) for _ in range(2 * _PNBUF)]
            + [pltpu.VMEM((EBLK, 16), jnp.float32),
               pltpu.VMEM((EBLK, 16), jnp.float32)]
            + [pltpu.SemaphoreType.DMA for _ in range(2 * _PNBUF)]
        ),
    )
    def pair(p1_hbm, p2_hbm, i0_2d, i1_2d, z_hbm, b0, b1,
             ga0, ga1, ga2, ga3, gb0, gb1, gb2, gb3, mbuf, sbuf,
             sa0, sa1, sa2, sa3, sb0, sb1, sb2, sb3):
        g1s = (ga0, ga1, ga2, ga3)
        g2s = (gb0, gb1, gb2, gb3)
        sem1s = (sa0, sa1, sa2, sa3)
        sem2s = (sb0, sb1, sb2, sb3)
        c = lax.axis_index("c")
        s = lax.axis_index("s")
        w = s * 2 + c
        start = w * pw
        pltpu.sync_copy(i0_2d.at[pl.ds(start, pw)], b0)
        pltpu.sync_copy(i1_2d.at[pl.ds(start, pw)], b1)
        for b in range(_PNBUF):
            pltpu.async_copy(p1_hbm.at[b0.at[b]], g1s[b], sem1s[b])
            pltpu.async_copy(p2_hbm.at[b1.at[b]], g2s[b], sem2s[b])

        def body(i, carry):
            for b in range(_PNBUF):
                j = i * _PNBUF + b
                g1, g2 = g1s[b], g2s[b]
                pltpu.make_async_copy(p1_hbm.at[b0.at[j]], g1, sem1s[b]).wait()
                pltpu.make_async_copy(p2_hbm.at[b1.at[j]], g2, sem2s[b]).wait()

                def row(r, carry2):
                    a0 = g1[r, pl.ds(0, 16)] + g2[r, pl.ds(0, 16)]
                    a1 = g1[r, pl.ds(16, 16)] + g2[r, pl.ds(16, 16)]
                    a2 = g1[r, pl.ds(32, 16)] + g2[r, pl.ds(32, 16)]
                    g1[r, pl.ds(0, 16)] = a0
                    g1[r, pl.ds(16, 16)] = a1
                    g1[r, pl.ds(32, 16)] = a2
                    m = jnp.max(jnp.maximum(jnp.maximum(a0, a1), a2))
                    mb = jnp.full((16,), m, dtype=jnp.float32)
                    e = (jnp.exp(a0 - mb) + jnp.exp(a1 - mb)
                         + jnp.exp(a2 - mb))
                    s = jnp.sum(e)
                    mbuf[r, :] = mb
                    sbuf[r, :] = jnp.full((16,), s, dtype=jnp.float32)
                    return carry2

                lax.fori_loop(0, EBLK, row, 0)

                def grp(g, carry2):
                    lane = jax.lax.broadcasted_iota(jnp.int32, (16,), 0)
                    rows16 = g * 16 + lane
                    mv = plsc.load_gather(mbuf, [rows16, lane])
                    sv = plsc.load_gather(sbuf, [rows16, lane])
                    bits = plsc.bitcast(sv, jnp.int32)
                    y = (bits - 1065353216).astype(jnp.float32) * 8.2629583e-08
                    for _ in range(3):
                        y = y - 1.0 + sv * jnp.exp(-y)
                    col40 = jnp.full((16,), 40, dtype=jnp.int32)
                    plsc.store_scatter(g1, [rows16, col40], mv + y)
                    return carry2

                lax.fori_loop(0, 8, grp, 0)
                pltpu.sync_copy(g1, z_hbm.at[pl.ds((start + j) * EBLK, EBLK)])
                jn = j + _PNBUF

                @pl.when(jn < pw)
                def _():
                    pltpu.async_copy(p1_hbm.at[b0.at[jn]], g1, sem1s[b])
                    pltpu.async_copy(p2_hbm.at[b1.at[jn]], g2, sem2s[b])
            return carry

        lax.fori_loop(0, pw // _PNBUF, body, 0)

    return pair


def _pad_edges(idx_a, idx_b, pad_a, pad_b, nbkp):
    ep = nbkp * EBLK
    e = idx_a.shape[0]
    a = jnp.concatenate([idx_a, pad_a[: ep - e]])
    b = jnp.concatenate([idx_b, pad_b[: ep - e]])
    return a.reshape(nbkp, EBLK), b.reshape(nbkp, EBLK)


def kernel(x, edge_index, ind_edge, in_degree, out_degree, masks, params):
    n, d = x.shape
    e = edge_index.shape[1]
    e_ind = ind_edge.shape[1]
    npad = _ceil_to(n + 16, NBLK)
    nbkp = _ceil_to(-(-e // EBLK), NW * 8)
    nbkp_i = _ceil_to(-(-e_ind // EBLK), NW * 8)

    f32 = jnp.float32
    pad_n = npad - n
    xp = jnp.pad(x, ((0, pad_n), (0, 0)))
    deg2 = jnp.pad(
        jnp.stack([out_degree, in_degree], axis=1), ((0, pad_n), (0, 0)))
    m4 = jnp.pad(
        jnp.stack([masks["out_deg_mask"], masks["out_deg_mask_bias"],
                   masks["in_deg_mask"], masks["in_deg_mask_bias"]], axis=1),
        ((0, pad_n), (0, 0)))

    npads = nbkp * EBLK - e
    dump = (n + (jnp.arange(npads, dtype=jnp.int32) % 16)).astype(jnp.int32)
    row2d, col2d = _pad_edges(edge_index[0], edge_index[1], dump, dump, nbkp)

    npads_i = nbkp_i * EBLK - e_ind
    spread_i = (jnp.arange(npads_i, dtype=jnp.int32) % n).astype(jnp.int32)
    i0_2d, i1_2d = _pad_edges(ind_edge[0], ind_edge[1], spread_i, spread_i,
                              nbkp_i)

    zeros_nd = jnp.zeros((npad, 128), f32)

    ls = params["layers"]

    def wpack(layer):
        b3 = jnp.zeros((8, 128), f32)
        b3 = b3.at[0].set(layer["b_sd"]).at[1].set(layer["b_ds"])
        b3 = b3.at[2].set(layer["b_fc"])
        b3 = b3.at[3, 0].set(layer["b_outf"][0]).at[3, 1].set(layer["b_inf"][0])
        to = _ceil_to(layer["out_tab"].shape[0], 8)
        ti = _ceil_to(layer["in_tab"].shape[0], 8)
        otab = jnp.pad(layer["out_tab"],
                       ((0, to - layer["out_tab"].shape[0]), (0, 0)))
        itab = jnp.pad(layer["in_tab"],
                       ((0, ti - layer["in_tab"].shape[0]), (0, 0)))
        return (otab, itab, layer["W_outf"], layer["W_inf"], layer["W_sd"],
                layer["W_ds"], layer["W_fc"], b3)

    w0 = wpack(ls[0])
    w1p = wpack(ls[1])
    wl1 = params["W_lin"][:128]
    wl2 = params["W_lin"][128:]
    blin = jnp.zeros((8, 40), f32).at[0].set(params["b_lin"])

    conv = _make_conv_kernel(npad, nbkp)
    pair = _make_pair_kernel(npad, nbkp_i)

    xs0, xo0, inv2 = _prep(xp, deg2)
    part_out0, part_in0 = conv(xs0, xo0, row2d, col2d, zeros_nd)
    x1, xs1, xo1, cp0 = _dense_layer(0, xp, part_out0, part_in0, inv2, deg2,
                                     m4, *w0)
    part_out1, part_in1 = conv(xs1, xo1, row2d, col2d, zeros_nd)
    p1, p2, cpf = _dense_layer(1, x1, part_out1, part_in1, inv2, deg2, m4,
                               *w1p, cp0=cp0, w1=wl1, w2=wl2, blin=blin)
    z = pair(p1, p2, i0_2d, i1_2d)
    logits = z[:e_ind, :40] - z[:e_ind, 40:41]
    c_ins = cpf[:n, 1:2]
    c_outs = cpf[:n, 0:1]
    return logits, c_ins, c_outs


# speedup vs baseline: 1.0005x; 1.0005x over previous
import functools

import jax
import jax.numpy as jnp
from jax import lax
from jax.experimental import pallas as pl
from jax.experimental.pallas import tpu as pltpu
from jax.experimental.pallas import tpu_sc as plsc

ALPHA_C = 0.5
NBLK = 1024
EBLK = 128
NW = 32
LSM_BLK = 4000


def _ceil_to(x, m):
    return (x + m - 1) // m * m


def _prep_body(x_ref, deg_ref, xs_ref, xo_ref, inv_ref):
    deg = deg_ref[...].astype(jnp.float32)
    inv = jnp.where(deg > 0, lax.rsqrt(jnp.maximum(deg, 1.0)), 0.0)
    x = x_ref[...]
    xo_ref[...] = inv[:, 0:1] * x
    xs_ref[...] = inv[:, 1:2] * x
    inv_ref[...] = inv


def _prep(xp, deg2):
    npad = xp.shape[0]
    grid = (npad // NBLK,)
    return pl.pallas_call(
        _prep_body,
        grid=grid,
        in_specs=[
            pl.BlockSpec((NBLK, 128), lambda i: (i, 0)),
            pl.BlockSpec((NBLK, 2), lambda i: (i, 0)),
        ],
        out_specs=[
            pl.BlockSpec((NBLK, 128), lambda i: (i, 0)),
            pl.BlockSpec((NBLK, 128), lambda i: (i, 0)),
            pl.BlockSpec((NBLK, 2), lambda i: (i, 0)),
        ],
        out_shape=[
            jax.ShapeDtypeStruct((npad, 128), jnp.float32),
            jax.ShapeDtypeStruct((npad, 128), jnp.float32),
            jax.ShapeDtypeStruct((npad, 2), jnp.float32),
        ],
    )(xp, deg2)


def _layer_out(x, sout, sin, inv, deg, m4, otab, itab, wof, wif,
               wsd, wds, wfc, b3):
    out_inv = inv[:, 0:1]
    in_inv = inv[:, 1:2]
    out_nei = out_inv * sout
    in_nei = in_inv * sin
    nb = x.shape[0]
    to = otab.shape[0]
    ti = itab.shape[0]
    oh_o = (lax.broadcasted_iota(jnp.int32, (nb, to), 1) == deg[:, 0:1]
            ).astype(jnp.float32)
    oh_i = (lax.broadcasted_iota(jnp.int32, (nb, ti), 1) == deg[:, 1:2]
            ).astype(jnp.float32)
    tgo = jnp.dot(oh_o, otab, preferred_element_type=jnp.float32)
    tgi = jnp.dot(oh_i, itab, preferred_element_type=jnp.float32)
    co = jnp.dot(out_nei - x + tgo, wof, preferred_element_type=jnp.float32)
    co = co + b3[3:4, 0:1]
    ci = jnp.dot(in_nei - x + tgi, wif, preferred_element_type=jnp.float32)
    ci = ci + b3[3:4, 1:2]
    c0 = 1.0 / (1.0 + jnp.exp(ci - co))
    c1 = 1.0 - c0
    c_out = c0 * m4[:, 0:1] + m4[:, 1:2]
    c_in = c1 * m4[:, 2:3] + m4[:, 3:4]
    h = (c_out * (jnp.dot(out_nei, wsd, preferred_element_type=jnp.float32)
                  + b3[0:1, :])
         + c_in * (jnp.dot(in_nei, wds, preferred_element_type=jnp.float32)
                   + b3[1:2, :])
         + ALPHA_C * (jnp.dot(x, wfc, preferred_element_type=jnp.float32)
                      + b3[2:3, :]))
    xn = jnp.maximum(h, 0.0)
    return xn, c_out, c_in


def _b0_body(x_ref, po0, po1, pi0, pi1, inv_ref, deg_ref, m4_ref,
             otab_ref, itab_ref, wof_ref, wif_ref, wsd_ref, wds_ref,
             wfc_ref, b3_ref, x1_ref, xs1_ref, xo1_ref, cp_ref):
    sout = po0[0] + po1[0]
    sin = pi0[0] + pi1[0]
    xn, c_out, c_in = _layer_out(
        x_ref[...], sout, sin, inv_ref[...], deg_ref[...], m4_ref[...],
        otab_ref[...], itab_ref[...], wof_ref[...], wif_ref[...],
        wsd_ref[...], wds_ref[...], wfc_ref[...], b3_ref[...])
    x1_ref[...] = xn
    inv = inv_ref[...]
    xo1_ref[...] = inv[:, 0:1] * xn
    xs1_ref[...] = inv[:, 1:2] * xn
    cp_ref[...] = jnp.concatenate([c_out, c_in], axis=1)


def _b1_body(x_ref, po0, po1, pi0, pi1, inv_ref, deg_ref, m4_ref,
             otab_ref, itab_ref, wof_ref, wif_ref, wsd_ref, wds_ref,
             wfc_ref, b3_ref, cp0_ref, w1_ref, w2_ref, blin_ref,
             p1_ref, p2_ref, cpf_ref):
    sout = po0[0] + po1[0]
    sin = pi0[0] + pi1[0]
    x1 = x_ref[...]
    xn, c_out, c_in = _layer_out(
        x1, sout, sin, inv_ref[...], deg_ref[...], m4_ref[...],
        otab_ref[...], itab_ref[...], wof_ref[...], wif_ref[...],
        wsd_ref[...], wds_ref[...], wfc_ref[...], b3_ref[...])
    xj = jnp.maximum(x1, xn)
    nb = xj.shape[0]
    p1 = jnp.dot(xj, w1_ref[...], preferred_element_type=jnp.float32)
    p1 = p1 + blin_ref[0:1, :]
    p2 = jnp.dot(xj, w2_ref[...], preferred_element_type=jnp.float32)
    p1_ref[...] = jnp.concatenate(
        [p1, jnp.full((nb, 8), -1e30, dtype=jnp.float32)], axis=1)
    p2_ref[...] = jnp.concatenate(
        [p2, jnp.zeros((nb, 8), dtype=jnp.float32)], axis=1)
    cpf_ref[...] = (cp0_ref[...] + jnp.concatenate([c_out, c_in], axis=1)) * 0.5


def _dense_layer(lidx, xp, part_out, part_in, inv2, deg2, m4,
                 otab, itab, wof, wif, wsd, wds, wfc, b3,
                 cp0=None, w1=None, w2=None, blin=None):
    npad = xp.shape[0]
    grid = (npad // NBLK,)
    to = otab.shape[0]
    ti = itab.shape[0]
    node_spec = lambda w: pl.BlockSpec((NBLK, w), lambda i: (i, 0))
    part_spec = lambda c: pl.BlockSpec((1, NBLK, 128), lambda i, c=c: (c, i, 0))
    full = lambda shape: pl.BlockSpec(shape, lambda i, s=shape: tuple(
        0 for _ in s))
    in_specs = [
        node_spec(128),
        part_spec(0), part_spec(1),
        part_spec(0), part_spec(1),
        node_spec(2), node_spec(2), node_spec(4),
        full((to, 128)), full((ti, 128)),
        full((128, 1)), full((128, 1)),
        full((128, 128)), full((128, 128)), full((128, 128)),
        full((8, 128)),
    ]
    args = [xp, part_out, part_out, part_in, part_in, inv2, deg2, m4,
            otab, itab, wof, wif, wsd, wds, wfc, b3]
    if lidx == 0:
        out_specs = [node_spec(128), node_spec(128), node_spec(128),
                     node_spec(2)]
        out_shape = [jax.ShapeDtypeStruct((npad, 128), jnp.float32)] * 3 + [
            jax.ShapeDtypeStruct((npad, 2), jnp.float32)]
        body = _b0_body
    else:
        in_specs += [node_spec(2), full((128, 40)), full((128, 40)),
                     full((8, 40))]
        args += [cp0, w1, w2, blin]
        out_specs = [node_spec(48), node_spec(48), node_spec(2)]
        out_shape = [jax.ShapeDtypeStruct((npad, 48), jnp.float32)] * 2 + [
            jax.ShapeDtypeStruct((npad, 2), jnp.float32)]
        body = _b1_body
    return pl.pallas_call(
        body, grid=grid, in_specs=in_specs, out_specs=out_specs,
        out_shape=out_shape)(*args)


def _lsm_body(z_ref, o_ref):
    z = z_ref[...]
    m = jnp.max(z, axis=1, keepdims=True)
    e = jnp.exp(z - m)
    s = jnp.sum(e, axis=1, keepdims=True)
    o_ref[...] = z[:, :40] - (m + jnp.log(s))


def _log_softmax(z, e_out):
    grid = (e_out // LSM_BLK,)
    return pl.pallas_call(
        _lsm_body,
        grid=grid,
        in_specs=[pl.BlockSpec((LSM_BLK, 48), lambda i: (i, 0))],
        out_specs=pl.BlockSpec((LSM_BLK, 40), lambda i: (i, 0)),
        out_shape=jax.ShapeDtypeStruct((e_out, 40), jnp.float32),
    )(z)


_NBUF = 2


@functools.lru_cache(maxsize=None)
def _make_conv_kernel(npad, nbkp):
    pw = nbkp // NW
    hw = pw // 2
    assert pw % 2 == 0 and hw % 8 == 0 and pw % _NBUF == 0
    rows_z = npad // 16
    mesh = plsc.VectorSubcoreMesh(core_axis_name="c", subcore_axis_name="s")

    @functools.partial(
        pl.kernel,
        mesh=mesh,
        name="conv_segsum",
        out_type=(
            jax.ShapeDtypeStruct((2, npad, 128), jnp.float32),
            jax.ShapeDtypeStruct((2, npad, 128), jnp.float32),
        ),
        scratch_types=(
            [pltpu.VMEM((pw, EBLK), jnp.int32),
             pltpu.VMEM((hw, EBLK), jnp.int32)]
            + [pltpu.VMEM((EBLK, 128), jnp.float32) for _ in range(_NBUF)]
            + [pltpu.VMEM_SHARED((npad, 128), jnp.float32)]
            + [pltpu.SemaphoreType.DMA for _ in range(_NBUF)]
        ),
    )
    def conv(xs_hbm, xo_hbm, row2d, col2d, zeros_hbm, sout_hbm, sin_hbm,
             gbuf, sbuf, d0, d1, acc, s0, s1):
        dbufs = (d0, d1)
        sems = (s0, s1)
        c = lax.axis_index("c")
        s = lax.axis_index("s")
        w = s * 2 + c
        start = w * pw

        def phase(table_hbm, g2d, s2d, out_prev, out_hbm):
            pltpu.sync_copy(g2d.at[pl.ds(start, pw)], gbuf)
            pltpu.sync_copy(s2d.at[pl.ds(start, hw)], sbuf)
            for b in range(_NBUF):
                pltpu.async_copy(table_hbm.at[gbuf.at[b]], dbufs[b], sems[b])
            if out_prev is not None:
                pltpu.sync_copy(acc.at[pl.ds(s * rows_z, rows_z)],
                                out_prev.at[c, pl.ds(s * rows_z, rows_z)])
            pltpu.sync_copy(zeros_hbm.at[pl.ds(s * rows_z, rows_z)],
                            acc.at[pl.ds(s * rows_z, rows_z)])
            plsc.subcore_barrier()

            def body(i, carry):
                for b in range(_NBUF):
                    j = i * _NBUF + b

                    @pl.when(j == hw)
                    def _():
                        pltpu.sync_copy(s2d.at[pl.ds(start + hw, hw)], sbuf)

                    jl = lax.select(j >= hw, j - hw, j)
                    pltpu.make_async_copy(table_hbm.at[gbuf.at[j]], dbufs[b],
                                          sems[b]).wait()
                    pltpu.sync_copy(dbufs[b], acc.at[sbuf.at[jl]], add=True)
                    jn = j + _NBUF

                    @pl.when(jn < pw)
                    def _():
                        pltpu.async_copy(table_hbm.at[gbuf.at[jn]], dbufs[b],
                                         sems[b])
                return carry

            lax.fori_loop(0, pw // _NBUF, body, 0)
            plsc.subcore_barrier()

        phase(xs_hbm, col2d, row2d, None, sout_hbm)
        phase(xo_hbm, row2d, col2d, sout_hbm, sin_hbm)
        pltpu.sync_copy(acc.at[pl.ds(s * rows_z, rows_z)],
                        sin_hbm.at[c, pl.ds(s * rows_z, rows_z)])

    return conv


_PNBUF = 4


@functools.lru_cache(maxsize=None)
def _make_pair_kernel(npad, nbkp):
    pw = nbkp // NW
    assert pw % _PNBUF == 0
    ep = nbkp * EBLK
    mesh = plsc.VectorSubcoreMesh(core_axis_name="c", subcore_axis_name="s")

    @functools.partial(
        pl.kernel,
        mesh=mesh,
        out_type=jax.ShapeDtypeStruct((ep, 48), jnp.float32),
        compiler_params=pltpu.CompilerParams(use_tc_tiling_on_sc=False,
                                             needs_layout_passes=False),
        scratch_types=(
            [pltpu.VMEM((pw, EBLK), jnp.int32),
             pltpu.VMEM((pw, EBLK), jnp.int32)]
            + [pltpu.VMEM((EBLK, 48), jnp.float32) for _ in range(2 * _PNBUF)]
            + [pltpu.VMEM((EBLK, 16), jnp.float32),
               pltpu.VMEM((EBLK, 16), jnp.float32)]
            + [pltpu.SemaphoreType.DMA for _ in range(2 * _PNBUF)]
        ),
    )
    def pair(p1_hbm, p2_hbm, i0_2d, i1_2d, z_hbm, b0, b1,
             ga0, ga1, ga2, ga3, gb0, gb1, gb2, gb3, mbuf, sbuf,
             sa0, sa1, sa2, sa3, sb0, sb1, sb2, sb3):
        g1s = (ga0, ga1, ga2, ga3)
        g2s = (gb0, gb1, gb2, gb3)
        sem1s = (sa0, sa1, sa2, sa3)
        sem2s = (sb0, sb1, sb2, sb3)
        c = lax.axis_index("c")
        s = lax.axis_index("s")
        w = s * 2 + c
        start = w * pw
        pltpu.sync_copy(i0_2d.at[pl.ds(start, pw)], b0)
        pltpu.sync_copy(i1_2d.at[pl.ds(start, pw)], b1)
        for b in range(_PNBUF):
            pltpu.async_copy(p1_hbm.at[b0.at[b]], g1s[b], sem1s[b])
            pltpu.async_copy(p2_hbm.at[b1.at[b]], g2s[b], sem2s[b])

        def body(i, carry):
            for b in range(_PNBUF):
                j = i * _PNBUF + b
                g1, g2 = g1s[b], g2s[b]
                pltpu.make_async_copy(p1_hbm.at[b0.at[j]], g1, sem1s[b]).wait()
                pltpu.make_async_copy(p2_hbm.at[b1.at[j]], g2, sem2s[b]).wait()

                def row(r4, carry2):
                    for rr in range(4):
                        r = r4 * 4 + rr
                        a0 = g1[r, pl.ds(0, 16)] + g2[r, pl.ds(0, 16)]
                        a1 = g1[r, pl.ds(16, 16)] + g2[r, pl.ds(16, 16)]
                        a2 = g1[r, pl.ds(32, 16)] + g2[r, pl.ds(32, 16)]
                        g1[r, pl.ds(0, 16)] = a0
                        g1[r, pl.ds(16, 16)] = a1
                        g1[r, pl.ds(32, 16)] = a2
                        m = jnp.max(jnp.maximum(jnp.maximum(a0, a1), a2))
                        mb = jnp.full((16,), m, dtype=jnp.float32)
                        e = (jnp.exp(a0 - mb) + jnp.exp(a1 - mb)
                             + jnp.exp(a2 - mb))
                        s = jnp.sum(e)
                        mbuf[r, :] = mb
                        sbuf[r, :] = jnp.full((16,), s, dtype=jnp.float32)
                    return carry2

                lax.fori_loop(0, EBLK // 4, row, 0)

                def grp(g, carry2):
                    lane = jax.lax.broadcasted_iota(jnp.int32, (16,), 0)
                    rows16 = g * 16 + lane
                    mv = plsc.load_gather(mbuf, [rows16, lane])
                    sv = plsc.load_gather(sbuf, [rows16, lane])
                    bits = plsc.bitcast(sv, jnp.int32)
                    y = (bits - 1065353216).astype(jnp.float32) * 8.2629583e-08
                    for _ in range(3):
                        y = y - 1.0 + sv * jnp.exp(-y)
                    col40 = jnp.full((16,), 40, dtype=jnp.int32)
                    plsc.store_scatter(g1, [rows16, col40], mv + y)
                    return carry2

                lax.fori_loop(0, 8, grp, 0)
                pltpu.sync_copy(g1, z_hbm.at[pl.ds((start + j) * EBLK, EBLK)])
                jn = j + _PNBUF

                @pl.when(jn < pw)
                def _():
                    pltpu.async_copy(p1_hbm.at[b0.at[jn]], g1, sem1s[b])
                    pltpu.async_copy(p2_hbm.at[b1.at[jn]], g2, sem2s[b])
            return carry

        lax.fori_loop(0, pw // _PNBUF, body, 0)

    return pair


def _pad_edges(idx_a, idx_b, pad_a, pad_b, nbkp):
    ep = nbkp * EBLK
    e = idx_a.shape[0]
    a = jnp.concatenate([idx_a, pad_a[: ep - e]])
    b = jnp.concatenate([idx_b, pad_b[: ep - e]])
    return a.reshape(nbkp, EBLK), b.reshape(nbkp, EBLK)


def kernel(x, edge_index, ind_edge, in_degree, out_degree, masks, params):
    n, d = x.shape
    e = edge_index.shape[1]
    e_ind = ind_edge.shape[1]
    npad = _ceil_to(n + 16, NBLK)
    nbkp = _ceil_to(-(-e // EBLK), NW * 8)
    nbkp_i = _ceil_to(-(-e_ind // EBLK), NW * 8)

    f32 = jnp.float32
    pad_n = npad - n
    xp = jnp.pad(x, ((0, pad_n), (0, 0)))
    deg2 = jnp.pad(
        jnp.stack([out_degree, in_degree], axis=1), ((0, pad_n), (0, 0)))
    m4 = jnp.pad(
        jnp.stack([masks["out_deg_mask"], masks["out_deg_mask_bias"],
                   masks["in_deg_mask"], masks["in_deg_mask_bias"]], axis=1),
        ((0, pad_n), (0, 0)))

    npads = nbkp * EBLK - e
    dump = (n + (jnp.arange(npads, dtype=jnp.int32) % 16)).astype(jnp.int32)
    row2d, col2d = _pad_edges(edge_index[0], edge_index[1], dump, dump, nbkp)

    npads_i = nbkp_i * EBLK - e_ind
    spread_i = (jnp.arange(npads_i, dtype=jnp.int32) % n).astype(jnp.int32)
    i0_2d, i1_2d = _pad_edges(ind_edge[0], ind_edge[1], spread_i, spread_i,
                              nbkp_i)

    zeros_nd = jnp.zeros((npad, 128), f32)

    ls = params["layers"]

    def wpack(layer):
        b3 = jnp.zeros((8, 128), f32)
        b3 = b3.at[0].set(layer["b_sd"]).at[1].set(layer["b_ds"])
        b3 = b3.at[2].set(layer["b_fc"])
        b3 = b3.at[3, 0].set(layer["b_outf"][0]).at[3, 1].set(layer["b_inf"][0])
        to = _ceil_to(layer["out_tab"].shape[0], 8)
        ti = _ceil_to(layer["in_tab"].shape[0], 8)
        otab = jnp.pad(layer["out_tab"],
                       ((0, to - layer["out_tab"].shape[0]), (0, 0)))
        itab = jnp.pad(layer["in_tab"],
                       ((0, ti - layer["in_tab"].shape[0]), (0, 0)))
        return (otab, itab, layer["W_outf"], layer["W_inf"], layer["W_sd"],
                layer["W_ds"], layer["W_fc"], b3)

    w0 = wpack(ls[0])
    w1p = wpack(ls[1])
    wl1 = params["W_lin"][:128]
    wl2 = params["W_lin"][128:]
    blin = jnp.zeros((8, 40), f32).at[0].set(params["b_lin"])

    conv = _make_conv_kernel(npad, nbkp)
    pair = _make_pair_kernel(npad, nbkp_i)

    xs0, xo0, inv2 = _prep(xp, deg2)
    part_out0, part_in0 = conv(xs0, xo0, row2d, col2d, zeros_nd)
    x1, xs1, xo1, cp0 = _dense_layer(0, xp, part_out0, part_in0, inv2, deg2,
                                     m4, *w0)
    part_out1, part_in1 = conv(xs1, xo1, row2d, col2d, zeros_nd)
    p1, p2, cpf = _dense_layer(1, x1, part_out1, part_in1, inv2, deg2, m4,
                               *w1p, cp0=cp0, w1=wl1, w2=wl2, blin=blin)
    z = pair(p1, p2, i0_2d, i1_2d)
    logits = z[:e_ind, :40] - z[:e_ind, 40:41]
    c_ins = cpf[:n, 1:2]
    c_outs = cpf[:n, 0:1]
    return logits, c_ins, c_outs


# speedup vs baseline: 1.2675x; 1.2668x over previous
import functools

import jax
import jax.numpy as jnp
from jax import lax
from jax.experimental import pallas as pl
from jax.experimental.pallas import tpu as pltpu
from jax.experimental.pallas import tpu_sc as plsc

ALPHA_C = 0.5
NBLK = 1024
EBLK = 128
NW = 32
LSM_BLK = 4000


def _ceil_to(x, m):
    return (x + m - 1) // m * m


def _prep_body(x_ref, deg_ref, xs_ref, xo_ref, inv_ref):
    deg = deg_ref[...].astype(jnp.float32)
    inv = jnp.where(deg > 0, lax.rsqrt(jnp.maximum(deg, 1.0)), 0.0)
    x = x_ref[...]
    xo_ref[...] = inv[:, 0:1] * x
    xs_ref[...] = inv[:, 1:2] * x
    inv_ref[...] = inv


def _prep(xp, deg2):
    npad = xp.shape[0]
    grid = (npad // NBLK,)
    return pl.pallas_call(
        _prep_body,
        grid=grid,
        in_specs=[
            pl.BlockSpec((NBLK, 128), lambda i: (i, 0)),
            pl.BlockSpec((NBLK, 2), lambda i: (i, 0)),
        ],
        out_specs=[
            pl.BlockSpec((NBLK, 128), lambda i: (i, 0)),
            pl.BlockSpec((NBLK, 128), lambda i: (i, 0)),
            pl.BlockSpec((NBLK, 2), lambda i: (i, 0)),
        ],
        out_shape=[
            jax.ShapeDtypeStruct((npad, 128), jnp.float32),
            jax.ShapeDtypeStruct((npad, 128), jnp.float32),
            jax.ShapeDtypeStruct((npad, 2), jnp.float32),
        ],
    )(xp, deg2)


def _layer_out(x, sout, sin, inv, deg, m4, otab, itab, wof, wif,
               wsd, wds, wfc, b3):
    out_inv = inv[:, 0:1]
    in_inv = inv[:, 1:2]
    out_nei = out_inv * sout
    in_nei = in_inv * sin
    nb = x.shape[0]
    to = otab.shape[0]
    ti = itab.shape[0]
    oh_o = (lax.broadcasted_iota(jnp.int32, (nb, to), 1) == deg[:, 0:1]
            ).astype(jnp.float32)
    oh_i = (lax.broadcasted_iota(jnp.int32, (nb, ti), 1) == deg[:, 1:2]
            ).astype(jnp.float32)
    tgo = jnp.dot(oh_o, otab, preferred_element_type=jnp.float32)
    tgi = jnp.dot(oh_i, itab, preferred_element_type=jnp.float32)
    co = jnp.dot(out_nei - x + tgo, wof, preferred_element_type=jnp.float32)
    co = co + b3[3:4, 0:1]
    ci = jnp.dot(in_nei - x + tgi, wif, preferred_element_type=jnp.float32)
    ci = ci + b3[3:4, 1:2]
    c0 = 1.0 / (1.0 + jnp.exp(ci - co))
    c1 = 1.0 - c0
    c_out = c0 * m4[:, 0:1] + m4[:, 1:2]
    c_in = c1 * m4[:, 2:3] + m4[:, 3:4]
    h = (c_out * (jnp.dot(out_nei, wsd, preferred_element_type=jnp.float32)
                  + b3[0:1, :])
         + c_in * (jnp.dot(in_nei, wds, preferred_element_type=jnp.float32)
                   + b3[1:2, :])
         + ALPHA_C * (jnp.dot(x, wfc, preferred_element_type=jnp.float32)
                      + b3[2:3, :]))
    xn = jnp.maximum(h, 0.0)
    return xn, c_out, c_in


def _b0_body(x_ref, po0, po1, pi0, pi1, inv_ref, deg_ref, m4_ref,
             otab_ref, itab_ref, wof_ref, wif_ref, wsd_ref, wds_ref,
             wfc_ref, b3_ref, x1_ref, xs1_ref, xo1_ref, cp_ref):
    sout = po0[0] + po1[0]
    sin = pi0[0] + pi1[0]
    xn, c_out, c_in = _layer_out(
        x_ref[...], sout, sin, inv_ref[...], deg_ref[...], m4_ref[...],
        otab_ref[...], itab_ref[...], wof_ref[...], wif_ref[...],
        wsd_ref[...], wds_ref[...], wfc_ref[...], b3_ref[...])
    x1_ref[...] = xn
    inv = inv_ref[...]
    xo1_ref[...] = inv[:, 0:1] * xn
    xs1_ref[...] = inv[:, 1:2] * xn
    cp_ref[...] = jnp.concatenate([c_out, c_in], axis=1)


def _b1_body(x_ref, po0, po1, pi0, pi1, inv_ref, deg_ref, m4_ref,
             otab_ref, itab_ref, wof_ref, wif_ref, wsd_ref, wds_ref,
             wfc_ref, b3_ref, cp0_ref, w1_ref, w2_ref, blin_ref,
             p1_ref, p2_ref, cpf_ref):
    sout = po0[0] + po1[0]
    sin = pi0[0] + pi1[0]
    x1 = x_ref[...]
    xn, c_out, c_in = _layer_out(
        x1, sout, sin, inv_ref[...], deg_ref[...], m4_ref[...],
        otab_ref[...], itab_ref[...], wof_ref[...], wif_ref[...],
        wsd_ref[...], wds_ref[...], wfc_ref[...], b3_ref[...])
    xj = jnp.maximum(x1, xn)
    nb = xj.shape[0]
    p1 = jnp.dot(xj, w1_ref[...], preferred_element_type=jnp.float32)
    p1 = p1 + blin_ref[0:1, :]
    p2 = jnp.dot(xj, w2_ref[...], preferred_element_type=jnp.float32)
    p1_ref[...] = jnp.concatenate(
        [p1, jnp.full((nb, 8), -1e30, dtype=jnp.float32)], axis=1)
    p2_ref[...] = jnp.concatenate(
        [p2, jnp.zeros((nb, 8), dtype=jnp.float32)], axis=1)
    cpf_ref[...] = (cp0_ref[...] + jnp.concatenate([c_out, c_in], axis=1)) * 0.5


def _dense_layer(lidx, xp, part_out, part_in, inv2, deg2, m4,
                 otab, itab, wof, wif, wsd, wds, wfc, b3,
                 cp0=None, w1=None, w2=None, blin=None):
    npad = xp.shape[0]
    grid = (npad // NBLK,)
    to = otab.shape[0]
    ti = itab.shape[0]
    node_spec = lambda w: pl.BlockSpec((NBLK, w), lambda i: (i, 0))
    part_spec = lambda c: pl.BlockSpec((1, NBLK, 128), lambda i, c=c: (c, i, 0))
    full = lambda shape: pl.BlockSpec(shape, lambda i, s=shape: tuple(
        0 for _ in s))
    in_specs = [
        node_spec(128),
        part_spec(0), part_spec(1),
        part_spec(0), part_spec(1),
        node_spec(2), node_spec(2), node_spec(4),
        full((to, 128)), full((ti, 128)),
        full((128, 1)), full((128, 1)),
        full((128, 128)), full((128, 128)), full((128, 128)),
        full((8, 128)),
    ]
    args = [xp, part_out, part_out, part_in, part_in, inv2, deg2, m4,
            otab, itab, wof, wif, wsd, wds, wfc, b3]
    if lidx == 0:
        out_specs = [node_spec(128), node_spec(128), node_spec(128),
                     node_spec(2)]
        out_shape = [jax.ShapeDtypeStruct((npad, 128), jnp.float32)] * 3 + [
            jax.ShapeDtypeStruct((npad, 2), jnp.float32)]
        body = _b0_body
    else:
        in_specs += [node_spec(2), full((128, 40)), full((128, 40)),
                     full((8, 40))]
        args += [cp0, w1, w2, blin]
        out_specs = [node_spec(48), node_spec(48), node_spec(2)]
        out_shape = [jax.ShapeDtypeStruct((npad, 48), jnp.float32)] * 2 + [
            jax.ShapeDtypeStruct((npad, 2), jnp.float32)]
        body = _b1_body
    return pl.pallas_call(
        body, grid=grid, in_specs=in_specs, out_specs=out_specs,
        out_shape=out_shape)(*args)


def _lsm_body(z_ref, o_ref):
    z = z_ref[...]
    m = jnp.max(z, axis=1, keepdims=True)
    e = jnp.exp(z - m)
    s = jnp.sum(e, axis=1, keepdims=True)
    o_ref[...] = z[:, :40] - (m + jnp.log(s))


def _log_softmax(z, e_out):
    grid = (e_out // LSM_BLK,)
    return pl.pallas_call(
        _lsm_body,
        grid=grid,
        in_specs=[pl.BlockSpec((LSM_BLK, 48), lambda i: (i, 0))],
        out_specs=pl.BlockSpec((LSM_BLK, 40), lambda i: (i, 0)),
        out_shape=jax.ShapeDtypeStruct((e_out, 40), jnp.float32),
    )(z)


_NBUF = 2


@functools.lru_cache(maxsize=None)
def _make_conv_kernel(npad, nbkp):
    pw = nbkp // NW
    hw = pw // 2
    assert pw % 2 == 0 and hw % 8 == 0 and pw % _NBUF == 0
    rows_z = npad // 16
    mesh = plsc.VectorSubcoreMesh(core_axis_name="c", subcore_axis_name="s")

    @functools.partial(
        pl.kernel,
        mesh=mesh,
        name="conv_segsum",
        out_type=(
            jax.ShapeDtypeStruct((2, npad, 128), jnp.float32),
            jax.ShapeDtypeStruct((2, npad, 128), jnp.float32),
        ),
        scratch_types=(
            [pltpu.VMEM((pw, EBLK), jnp.int32),
             pltpu.VMEM((hw, EBLK), jnp.int32)]
            + [pltpu.VMEM((EBLK, 128), jnp.float32) for _ in range(_NBUF)]
            + [pltpu.VMEM_SHARED((npad, 128), jnp.float32)]
            + [pltpu.SemaphoreType.DMA for _ in range(_NBUF)]
        ),
    )
    def conv(xs_hbm, xo_hbm, row2d, col2d, zeros_hbm, sout_hbm, sin_hbm,
             gbuf, sbuf, d0, d1, acc, s0, s1):
        dbufs = (d0, d1)
        sems = (s0, s1)
        c = lax.axis_index("c")
        s = lax.axis_index("s")
        w = s * 2 + c
        start = w * pw

        def phase(table_hbm, g2d, s2d, out_prev, out_hbm):
            pltpu.sync_copy(g2d.at[pl.ds(start, pw)], gbuf)
            pltpu.sync_copy(s2d.at[pl.ds(start, hw)], sbuf)
            for b in range(_NBUF):
                pltpu.async_copy(table_hbm.at[gbuf.at[b]], dbufs[b], sems[b])
            if out_prev is not None:
                pltpu.sync_copy(acc.at[pl.ds(s * rows_z, rows_z)],
                                out_prev.at[c, pl.ds(s * rows_z, rows_z)])
            pltpu.sync_copy(zeros_hbm.at[pl.ds(s * rows_z, rows_z)],
                            acc.at[pl.ds(s * rows_z, rows_z)])
            plsc.subcore_barrier()

            def body(i, carry):
                for b in range(_NBUF):
                    j = i * _NBUF + b

                    @pl.when(j == hw)
                    def _():
                        pltpu.sync_copy(s2d.at[pl.ds(start + hw, hw)], sbuf)

                    jl = lax.select(j >= hw, j - hw, j)
                    pltpu.make_async_copy(table_hbm.at[gbuf.at[j]], dbufs[b],
                                          sems[b]).wait()
                    pltpu.sync_copy(dbufs[b], acc.at[sbuf.at[jl]], add=True)
                    jn = j + _NBUF

                    @pl.when(jn < pw)
                    def _():
                        pltpu.async_copy(table_hbm.at[gbuf.at[jn]], dbufs[b],
                                         sems[b])
                return carry

            lax.fori_loop(0, pw // _NBUF, body, 0)
            plsc.subcore_barrier()

        phase(xs_hbm, col2d, row2d, None, sout_hbm)
        phase(xo_hbm, row2d, col2d, sout_hbm, sin_hbm)
        pltpu.sync_copy(acc.at[pl.ds(s * rows_z, rows_z)],
                        sin_hbm.at[c, pl.ds(s * rows_z, rows_z)])

    return conv


_PNBUF = 4


@functools.lru_cache(maxsize=None)
def _make_pair_kernel(npad, nbkp):
    pw = nbkp // NW
    assert pw % _PNBUF == 0
    ep = nbkp * EBLK
    mesh = plsc.VectorSubcoreMesh(core_axis_name="c", subcore_axis_name="s")

    @functools.partial(
        pl.kernel,
        mesh=mesh,
        out_type=jax.ShapeDtypeStruct((ep, 48), jnp.float32),
        compiler_params=pltpu.CompilerParams(use_tc_tiling_on_sc=False,
                                             needs_layout_passes=False),
        scratch_types=(
            [pltpu.VMEM((pw, EBLK), jnp.int32),
             pltpu.VMEM((pw, EBLK), jnp.int32)]
            + [pltpu.VMEM((EBLK, 48), jnp.float32) for _ in range(2 * _PNBUF)]
            + [pltpu.VMEM((EBLK, 16), jnp.float32),
               pltpu.VMEM((EBLK, 16), jnp.float32)]
            + [pltpu.SemaphoreType.DMA for _ in range(2 * _PNBUF)]
        ),
    )
    def pair(p1_hbm, p2_hbm, i0_2d, i1_2d, z_hbm, b0, b1,
             ga0, ga1, ga2, ga3, gb0, gb1, gb2, gb3, mbuf, sbuf,
             sa0, sa1, sa2, sa3, sb0, sb1, sb2, sb3):
        g1s = (ga0, ga1, ga2, ga3)
        g2s = (gb0, gb1, gb2, gb3)
        sem1s = (sa0, sa1, sa2, sa3)
        sem2s = (sb0, sb1, sb2, sb3)
        c = lax.axis_index("c")
        s = lax.axis_index("s")
        w = s * 2 + c
        start = w * pw
        pltpu.sync_copy(i0_2d.at[pl.ds(start, pw)], b0)
        pltpu.sync_copy(i1_2d.at[pl.ds(start, pw)], b1)
        for b in range(_PNBUF):
            pltpu.async_copy(p1_hbm.at[b0.at[b]], g1s[b], sem1s[b])
            pltpu.async_copy(p2_hbm.at[b1.at[b]], g2s[b], sem2s[b])

        def body(i, carry):
            for b in range(_PNBUF):
                j = i * _PNBUF + b
                g1, g2 = g1s[b], g2s[b]
                pltpu.make_async_copy(p1_hbm.at[b0.at[j]], g1, sem1s[b]).wait()
                pltpu.make_async_copy(p2_hbm.at[b1.at[j]], g2, sem2s[b]).wait()

                @plsc.parallel_loop(0, EBLK, unroll=4)
                def _row(r):
                    a0 = g1[r, pl.ds(0, 16)] + g2[r, pl.ds(0, 16)]
                    a1 = g1[r, pl.ds(16, 16)] + g2[r, pl.ds(16, 16)]
                    a2 = g1[r, pl.ds(32, 16)] + g2[r, pl.ds(32, 16)]
                    g1[r, pl.ds(0, 16)] = a0
                    g1[r, pl.ds(16, 16)] = a1
                    g1[r, pl.ds(32, 16)] = a2
                    m = jnp.max(jnp.maximum(jnp.maximum(a0, a1), a2))
                    mb = jnp.full((16,), m, dtype=jnp.float32)
                    e = (jnp.exp(a0 - mb) + jnp.exp(a1 - mb)
                         + jnp.exp(a2 - mb))
                    s = jnp.sum(e)
                    mbuf[r, :] = mb
                    sbuf[r, :] = jnp.full((16,), s, dtype=jnp.float32)

                def grp(g, carry2):
                    lane = jax.lax.broadcasted_iota(jnp.int32, (16,), 0)
                    rows16 = g * 16 + lane
                    mv = plsc.load_gather(mbuf, [rows16, lane])
                    sv = plsc.load_gather(sbuf, [rows16, lane])
                    bits = plsc.bitcast(sv, jnp.int32)
                    y = (bits - 1065353216).astype(jnp.float32) * 8.2629583e-08
                    for _ in range(3):
                        y = y - 1.0 + sv * jnp.exp(-y)
                    col40 = jnp.full((16,), 40, dtype=jnp.int32)
                    plsc.store_scatter(g1, [rows16, col40], mv + y)
                    return carry2

                lax.fori_loop(0, 8, grp, 0)
                pltpu.sync_copy(g1, z_hbm.at[pl.ds((start + j) * EBLK, EBLK)])
                jn = j + _PNBUF

                @pl.when(jn < pw)
                def _():
                    pltpu.async_copy(p1_hbm.at[b0.at[jn]], g1, sem1s[b])
                    pltpu.async_copy(p2_hbm.at[b1.at[jn]], g2, sem2s[b])
            return carry

        lax.fori_loop(0, pw // _PNBUF, body, 0)

    return pair


def _pad_edges(idx_a, idx_b, pad_a, pad_b, nbkp):
    ep = nbkp * EBLK
    e = idx_a.shape[0]
    a = jnp.concatenate([idx_a, pad_a[: ep - e]])
    b = jnp.concatenate([idx_b, pad_b[: ep - e]])
    return a.reshape(nbkp, EBLK), b.reshape(nbkp, EBLK)


def kernel(x, edge_index, ind_edge, in_degree, out_degree, masks, params):
    n, d = x.shape
    e = edge_index.shape[1]
    e_ind = ind_edge.shape[1]
    npad = _ceil_to(n + 16, NBLK)
    nbkp = _ceil_to(-(-e // EBLK), NW * 8)
    nbkp_i = _ceil_to(-(-e_ind // EBLK), NW * 8)

    f32 = jnp.float32
    pad_n = npad - n
    xp = jnp.pad(x, ((0, pad_n), (0, 0)))
    deg2 = jnp.pad(
        jnp.stack([out_degree, in_degree], axis=1), ((0, pad_n), (0, 0)))
    m4 = jnp.pad(
        jnp.stack([masks["out_deg_mask"], masks["out_deg_mask_bias"],
                   masks["in_deg_mask"], masks["in_deg_mask_bias"]], axis=1),
        ((0, pad_n), (0, 0)))

    npads = nbkp * EBLK - e
    dump = (n + (jnp.arange(npads, dtype=jnp.int32) % 16)).astype(jnp.int32)
    row2d, col2d = _pad_edges(edge_index[0], edge_index[1], dump, dump, nbkp)

    npads_i = nbkp_i * EBLK - e_ind
    spread_i = (jnp.arange(npads_i, dtype=jnp.int32) % n).astype(jnp.int32)
    i0_2d, i1_2d = _pad_edges(ind_edge[0], ind_edge[1], spread_i, spread_i,
                              nbkp_i)

    zeros_nd = jnp.zeros((npad, 128), f32)

    ls = params["layers"]

    def wpack(layer):
        b3 = jnp.zeros((8, 128), f32)
        b3 = b3.at[0].set(layer["b_sd"]).at[1].set(layer["b_ds"])
        b3 = b3.at[2].set(layer["b_fc"])
        b3 = b3.at[3, 0].set(layer["b_outf"][0]).at[3, 1].set(layer["b_inf"][0])
        to = _ceil_to(layer["out_tab"].shape[0], 8)
        ti = _ceil_to(layer["in_tab"].shape[0], 8)
        otab = jnp.pad(layer["out_tab"],
                       ((0, to - layer["out_tab"].shape[0]), (0, 0)))
        itab = jnp.pad(layer["in_tab"],
                       ((0, ti - layer["in_tab"].shape[0]), (0, 0)))
        return (otab, itab, layer["W_outf"], layer["W_inf"], layer["W_sd"],
                layer["W_ds"], layer["W_fc"], b3)

    w0 = wpack(ls[0])
    w1p = wpack(ls[1])
    wl1 = params["W_lin"][:128]
    wl2 = params["W_lin"][128:]
    blin = jnp.zeros((8, 40), f32).at[0].set(params["b_lin"])

    conv = _make_conv_kernel(npad, nbkp)
    pair = _make_pair_kernel(npad, nbkp_i)

    xs0, xo0, inv2 = _prep(xp, deg2)
    part_out0, part_in0 = conv(xs0, xo0, row2d, col2d, zeros_nd)
    x1, xs1, xo1, cp0 = _dense_layer(0, xp, part_out0, part_in0, inv2, deg2,
                                     m4, *w0)
    part_out1, part_in1 = conv(xs1, xo1, row2d, col2d, zeros_nd)
    p1, p2, cpf = _dense_layer(1, x1, part_out1, part_in1, inv2, deg2, m4,
                               *w1p, cp0=cp0, w1=wl1, w2=wl2, blin=blin)
    z = pair(p1, p2, i0_2d, i1_2d)
    logits = z[:e_ind, :40] - z[:e_ind, 40:41]
    c_ins = cpf[:n, 1:2]
    c_outs = cpf[:n, 0:1]
    return logits, c_ins, c_outs


# speedup vs baseline: 1.3878x; 1.0949x over previous
import functools

import jax
import jax.numpy as jnp
from jax import lax
from jax.experimental import pallas as pl
from jax.experimental.pallas import tpu as pltpu
from jax.experimental.pallas import tpu_sc as plsc

ALPHA_C = 0.5
NBLK = 1024
EBLK = 128
NW = 32
LSM_BLK = 4000


def _ceil_to(x, m):
    return (x + m - 1) // m * m


def _prep_body(x_ref, deg_ref, xs_ref, xo_ref, inv_ref):
    deg = deg_ref[...].astype(jnp.float32)
    inv = jnp.where(deg > 0, lax.rsqrt(jnp.maximum(deg, 1.0)), 0.0)
    x = x_ref[...]
    xo_ref[...] = inv[:, 0:1] * x
    xs_ref[...] = inv[:, 1:2] * x
    inv_ref[...] = inv


def _prep(xp, deg2):
    npad = xp.shape[0]
    grid = (npad // NBLK,)
    return pl.pallas_call(
        _prep_body,
        grid=grid,
        in_specs=[
            pl.BlockSpec((NBLK, 128), lambda i: (i, 0)),
            pl.BlockSpec((NBLK, 2), lambda i: (i, 0)),
        ],
        out_specs=[
            pl.BlockSpec((NBLK, 128), lambda i: (i, 0)),
            pl.BlockSpec((NBLK, 128), lambda i: (i, 0)),
            pl.BlockSpec((NBLK, 2), lambda i: (i, 0)),
        ],
        out_shape=[
            jax.ShapeDtypeStruct((npad, 128), jnp.float32),
            jax.ShapeDtypeStruct((npad, 128), jnp.float32),
            jax.ShapeDtypeStruct((npad, 2), jnp.float32),
        ],
    )(xp, deg2)


def _layer_out(x, sout, sin, inv, deg, m4, otab, itab, wof, wif,
               wsd, wds, wfc, b3):
    out_inv = inv[:, 0:1]
    in_inv = inv[:, 1:2]
    out_nei = out_inv * sout
    in_nei = in_inv * sin
    nb = x.shape[0]
    to = otab.shape[0]
    ti = itab.shape[0]
    oh_o = (lax.broadcasted_iota(jnp.int32, (nb, to), 1) == deg[:, 0:1]
            ).astype(jnp.float32)
    oh_i = (lax.broadcasted_iota(jnp.int32, (nb, ti), 1) == deg[:, 1:2]
            ).astype(jnp.float32)
    tgo = jnp.dot(oh_o, otab, preferred_element_type=jnp.float32)
    tgi = jnp.dot(oh_i, itab, preferred_element_type=jnp.float32)
    co = jnp.dot(out_nei - x + tgo, wof, preferred_element_type=jnp.float32)
    co = co + b3[3:4, 0:1]
    ci = jnp.dot(in_nei - x + tgi, wif, preferred_element_type=jnp.float32)
    ci = ci + b3[3:4, 1:2]
    c0 = 1.0 / (1.0 + jnp.exp(ci - co))
    c1 = 1.0 - c0
    c_out = c0 * m4[:, 0:1] + m4[:, 1:2]
    c_in = c1 * m4[:, 2:3] + m4[:, 3:4]
    h = (c_out * (jnp.dot(out_nei, wsd, preferred_element_type=jnp.float32)
                  + b3[0:1, :])
         + c_in * (jnp.dot(in_nei, wds, preferred_element_type=jnp.float32)
                   + b3[1:2, :])
         + ALPHA_C * (jnp.dot(x, wfc, preferred_element_type=jnp.float32)
                      + b3[2:3, :]))
    xn = jnp.maximum(h, 0.0)
    return xn, c_out, c_in


def _b0_body(x_ref, po0, po1, pi0, pi1, inv_ref, deg_ref, m4_ref,
             otab_ref, itab_ref, wof_ref, wif_ref, wsd_ref, wds_ref,
             wfc_ref, b3_ref, x1_ref, xs1_ref, xo1_ref, cp_ref):
    sout = po0[0] + po1[0]
    sin = pi0[0] + pi1[0]
    xn, c_out, c_in = _layer_out(
        x_ref[...], sout, sin, inv_ref[...], deg_ref[...], m4_ref[...],
        otab_ref[...], itab_ref[...], wof_ref[...], wif_ref[...],
        wsd_ref[...], wds_ref[...], wfc_ref[...], b3_ref[...])
    x1_ref[...] = xn
    inv = inv_ref[...]
    xo1_ref[...] = inv[:, 0:1] * xn
    xs1_ref[...] = inv[:, 1:2] * xn
    cp_ref[...] = jnp.concatenate([c_out, c_in], axis=1)


def _b1_body(x_ref, po0, po1, pi0, pi1, inv_ref, deg_ref, m4_ref,
             otab_ref, itab_ref, wof_ref, wif_ref, wsd_ref, wds_ref,
             wfc_ref, b3_ref, cp0_ref, w1_ref, w2_ref, blin_ref,
             p1_ref, p2_ref, cpf_ref):
    sout = po0[0] + po1[0]
    sin = pi0[0] + pi1[0]
    x1 = x_ref[...]
    xn, c_out, c_in = _layer_out(
        x1, sout, sin, inv_ref[...], deg_ref[...], m4_ref[...],
        otab_ref[...], itab_ref[...], wof_ref[...], wif_ref[...],
        wsd_ref[...], wds_ref[...], wfc_ref[...], b3_ref[...])
    xj = jnp.maximum(x1, xn)
    nb = xj.shape[0]
    p1 = jnp.dot(xj, w1_ref[...], preferred_element_type=jnp.float32)
    p1 = p1 + blin_ref[0:1, :]
    p2 = jnp.dot(xj, w2_ref[...], preferred_element_type=jnp.float32)
    p1_ref[...] = jnp.concatenate(
        [p1, jnp.full((nb, 8), -1e30, dtype=jnp.float32)], axis=1)
    p2_ref[...] = jnp.concatenate(
        [p2, jnp.zeros((nb, 8), dtype=jnp.float32)], axis=1)
    cpf_ref[...] = (cp0_ref[...] + jnp.concatenate([c_out, c_in], axis=1)) * 0.5


def _dense_layer(lidx, xp, part_out, part_in, inv2, deg2, m4,
                 otab, itab, wof, wif, wsd, wds, wfc, b3,
                 cp0=None, w1=None, w2=None, blin=None):
    npad = xp.shape[0]
    grid = (npad // NBLK,)
    to = otab.shape[0]
    ti = itab.shape[0]
    node_spec = lambda w: pl.BlockSpec((NBLK, w), lambda i: (i, 0))
    part_spec = lambda c: pl.BlockSpec((1, NBLK, 128), lambda i, c=c: (c, i, 0))
    full = lambda shape: pl.BlockSpec(shape, lambda i, s=shape: tuple(
        0 for _ in s))
    in_specs = [
        node_spec(128),
        part_spec(0), part_spec(1),
        part_spec(0), part_spec(1),
        node_spec(2), node_spec(2), node_spec(4),
        full((to, 128)), full((ti, 128)),
        full((128, 1)), full((128, 1)),
        full((128, 128)), full((128, 128)), full((128, 128)),
        full((8, 128)),
    ]
    args = [xp, part_out, part_out, part_in, part_in, inv2, deg2, m4,
            otab, itab, wof, wif, wsd, wds, wfc, b3]
    if lidx == 0:
        out_specs = [node_spec(128), node_spec(128), node_spec(128),
                     node_spec(2)]
        out_shape = [jax.ShapeDtypeStruct((npad, 128), jnp.float32)] * 3 + [
            jax.ShapeDtypeStruct((npad, 2), jnp.float32)]
        body = _b0_body
    else:
        in_specs += [node_spec(2), full((128, 40)), full((128, 40)),
                     full((8, 40))]
        args += [cp0, w1, w2, blin]
        out_specs = [node_spec(48), node_spec(48), node_spec(2)]
        out_shape = [jax.ShapeDtypeStruct((npad, 48), jnp.float32)] * 2 + [
            jax.ShapeDtypeStruct((npad, 2), jnp.float32)]
        body = _b1_body
    return pl.pallas_call(
        body, grid=grid, in_specs=in_specs, out_specs=out_specs,
        out_shape=out_shape)(*args)


def _lsm_body(z_ref, o_ref):
    z = z_ref[...]
    m = jnp.max(z, axis=1, keepdims=True)
    e = jnp.exp(z - m)
    s = jnp.sum(e, axis=1, keepdims=True)
    o_ref[...] = z[:, :40] - (m + jnp.log(s))


def _log_softmax(z, e_out):
    grid = (e_out // LSM_BLK,)
    return pl.pallas_call(
        _lsm_body,
        grid=grid,
        in_specs=[pl.BlockSpec((LSM_BLK, 48), lambda i: (i, 0))],
        out_specs=pl.BlockSpec((LSM_BLK, 40), lambda i: (i, 0)),
        out_shape=jax.ShapeDtypeStruct((e_out, 40), jnp.float32),
    )(z)


_NBUF = 2


@functools.lru_cache(maxsize=None)
def _make_conv_kernel(npad, nbkp):
    pw = nbkp // NW
    hw = pw // 2
    assert pw % 2 == 0 and hw % 8 == 0 and pw % _NBUF == 0
    rows_z = npad // 16
    mesh = plsc.VectorSubcoreMesh(core_axis_name="c", subcore_axis_name="s")

    @functools.partial(
        pl.kernel,
        mesh=mesh,
        name="conv_segsum",
        out_type=(
            jax.ShapeDtypeStruct((2, npad, 128), jnp.float32),
            jax.ShapeDtypeStruct((2, npad, 128), jnp.float32),
        ),
        scratch_types=(
            [pltpu.VMEM((pw, EBLK), jnp.int32),
             pltpu.VMEM((hw, EBLK), jnp.int32)]
            + [pltpu.VMEM((EBLK, 128), jnp.float32) for _ in range(_NBUF)]
            + [pltpu.VMEM_SHARED((npad, 128), jnp.float32)]
            + [pltpu.SemaphoreType.DMA for _ in range(_NBUF)]
        ),
    )
    def conv(xs_hbm, xo_hbm, row2d, col2d, zeros_hbm, sout_hbm, sin_hbm,
             gbuf, sbuf, d0, d1, acc, s0, s1):
        dbufs = (d0, d1)
        sems = (s0, s1)
        c = lax.axis_index("c")
        s = lax.axis_index("s")
        w = s * 2 + c
        start = w * pw

        def phase(table_hbm, g2d, s2d, out_prev, out_hbm):
            pltpu.sync_copy(g2d.at[pl.ds(start, pw)], gbuf)
            pltpu.sync_copy(s2d.at[pl.ds(start, hw)], sbuf)
            for b in range(_NBUF):
                pltpu.async_copy(table_hbm.at[gbuf.at[b]], dbufs[b], sems[b])
            if out_prev is not None:
                pltpu.sync_copy(acc.at[pl.ds(s * rows_z, rows_z)],
                                out_prev.at[c, pl.ds(s * rows_z, rows_z)])
            pltpu.sync_copy(zeros_hbm.at[pl.ds(s * rows_z, rows_z)],
                            acc.at[pl.ds(s * rows_z, rows_z)])
            plsc.subcore_barrier()

            def body(i, carry):
                for b in range(_NBUF):
                    j = i * _NBUF + b

                    @pl.when(j == hw)
                    def _():
                        pltpu.sync_copy(s2d.at[pl.ds(start + hw, hw)], sbuf)

                    jl = lax.select(j >= hw, j - hw, j)
                    pltpu.make_async_copy(table_hbm.at[gbuf.at[j]], dbufs[b],
                                          sems[b]).wait()
                    pltpu.sync_copy(dbufs[b], acc.at[sbuf.at[jl]], add=True)
                    jn = j + _NBUF

                    @pl.when(jn < pw)
                    def _():
                        pltpu.async_copy(table_hbm.at[gbuf.at[jn]], dbufs[b],
                                         sems[b])
                return carry

            lax.fori_loop(0, pw // _NBUF, body, 0)
            plsc.subcore_barrier()

        phase(xs_hbm, col2d, row2d, None, sout_hbm)
        phase(xo_hbm, row2d, col2d, sout_hbm, sin_hbm)
        pltpu.sync_copy(acc.at[pl.ds(s * rows_z, rows_z)],
                        sin_hbm.at[c, pl.ds(s * rows_z, rows_z)])

    return conv


_PNBUF = 2


@functools.lru_cache(maxsize=None)
def _make_pair_kernel(npad, nbk, idx_rows):
    base = nbk // NW
    rem = nbk % NW
    pwm = base + (1 if rem else 0)
    niter = _ceil_to(pwm, _PNBUF) // _PNBUF
    mesh = plsc.VectorSubcoreMesh(core_axis_name="c", subcore_axis_name="s")

    @functools.partial(
        pl.kernel,
        mesh=mesh,
        name="pair_head",
        out_type=jax.ShapeDtypeStruct((5, nbk, 8, EBLK), jnp.float32),
        compiler_params=pltpu.CompilerParams(use_tc_tiling_on_sc=False,
                                             needs_layout_passes=False),
        scratch_types=(
            [pltpu.VMEM((pwm, EBLK), jnp.int32),
             pltpu.VMEM((pwm, EBLK), jnp.int32)]
            + [pltpu.VMEM((EBLK, 48), jnp.float32) for _ in range(2 * _PNBUF)]
            + [pltpu.VMEM((8, 40, 16), jnp.float32),
               pltpu.VMEM((40, EBLK), jnp.float32)]
            + [pltpu.SemaphoreType.DMA for _ in range(2 * _PNBUF)]
        ),
    )
    def pair(p1_hbm, p2_hbm, i0_2d, i1_2d, o4_hbm, b0, b1,
             ga0, ga1, gb0, gb1, ztb, obuf, sa0, sa1, sb0, sb1):
        g1s = (ga0, ga1)
        g2s = (gb0, gb1)
        sem1s = (sa0, sa1)
        sem2s = (sb0, sb1)
        c = lax.axis_index("c")
        s = lax.axis_index("s")
        w = s * 2 + c
        start = w * base + jnp.minimum(w, rem)
        cnt = base + (w < rem).astype(jnp.int32)
        pltpu.sync_copy(i0_2d.at[pl.ds(start, pwm)], b0)
        pltpu.sync_copy(i1_2d.at[pl.ds(start, pwm)], b1)
        for b in range(_PNBUF):
            pltpu.async_copy(p1_hbm.at[b0.at[b]], g1s[b], sem1s[b])
            pltpu.async_copy(p2_hbm.at[b1.at[b]], g2s[b], sem2s[b])

        def body(i, carry):
            for b in range(_PNBUF):
                j = i * _PNBUF + b
                g1, g2 = g1s[b], g2s[b]

                @pl.when(j < cnt)
                def _blk():
                    pltpu.make_async_copy(p1_hbm.at[b0.at[j]], g1,
                                          sem1s[b]).wait()
                    pltpu.make_async_copy(p2_hbm.at[b1.at[j]], g2,
                                          sem2s[b]).wait()

                    @plsc.parallel_loop(0, 8, unroll=2)
                    def _grp(g):
                        lane = jax.lax.broadcasted_iota(jnp.int32, (16,), 0)
                        rows16 = g * 16 + lane
                        neg = jnp.full((16,), -3e38, dtype=jnp.float32)
                        mp = [neg, neg, neg, neg]
                        for jj in range(40):
                            zj = (plsc.load_gather(
                                      g1, [rows16,
                                           jnp.full((16,), jj, jnp.int32)])
                                  + plsc.load_gather(
                                      g2, [rows16,
                                           jnp.full((16,), jj, jnp.int32)]))
                            ztb[g, jj, :] = zj
                            mp[jj % 4] = jnp.maximum(mp[jj % 4], zj)
                        m = jnp.maximum(jnp.maximum(mp[0], mp[1]),
                                        jnp.maximum(mp[2], mp[3]))
                        zero = jnp.zeros((16,), dtype=jnp.float32)
                        sp = [zero, zero, zero, zero]
                        for jj in range(40):
                            sp[jj % 4] = sp[jj % 4] + jnp.exp(ztb[g, jj, :] - m)
                        sv = (sp[0] + sp[1]) + (sp[2] + sp[3])
                        bits = plsc.bitcast(sv, jnp.int32)
                        y = ((bits - 1065353216).astype(jnp.float32)
                             * 8.2629583e-08)
                        for _ in range(3):
                            y = y - 1.0 + sv * jnp.exp(-y)
                        l = m + y
                        for jj in range(40):
                            obuf[jj, pl.ds(g * 16, 16)] = ztb[g, jj, :] - l

                    blk = start + j
                    for jt in range(5):
                        pltpu.sync_copy(obuf.at[pl.ds(jt * 8, 8)],
                                        o4_hbm.at[jt, blk])
                    jn = j + _PNBUF

                    @pl.when(jn < cnt)
                    def _():
                        pltpu.async_copy(p1_hbm.at[b0.at[jn]], g1, sem1s[b])
                        pltpu.async_copy(p2_hbm.at[b1.at[jn]], g2, sem2s[b])
            return carry

        lax.fori_loop(0, niter, body, 0)

    return pair


def _pad_edges(idx_a, idx_b, pad_a, pad_b, nbkp):
    ep = nbkp * EBLK
    e = idx_a.shape[0]
    a = jnp.concatenate([idx_a, pad_a[: ep - e]])
    b = jnp.concatenate([idx_b, pad_b[: ep - e]])
    return a.reshape(nbkp, EBLK), b.reshape(nbkp, EBLK)


def kernel(x, edge_index, ind_edge, in_degree, out_degree, masks, params):
    n, d = x.shape
    e = edge_index.shape[1]
    e_ind = ind_edge.shape[1]
    npad = _ceil_to(n + 16, NBLK)
    nbkp = _ceil_to(-(-e // EBLK), NW * 8)
    nbk_i = -(-e_ind // EBLK)
    assert e_ind % EBLK == 0
    pwm_i = nbk_i // NW + (1 if nbk_i % NW else 0)
    idx_rows_i = _ceil_to((NW - 1) * (nbk_i // NW) + (nbk_i % NW) + pwm_i, 8)

    f32 = jnp.float32
    pad_n = npad - n
    xp = jnp.pad(x, ((0, pad_n), (0, 0)))
    deg2 = jnp.pad(
        jnp.stack([out_degree, in_degree], axis=1), ((0, pad_n), (0, 0)))
    m4 = jnp.pad(
        jnp.stack([masks["out_deg_mask"], masks["out_deg_mask_bias"],
                   masks["in_deg_mask"], masks["in_deg_mask_bias"]], axis=1),
        ((0, pad_n), (0, 0)))

    npads = nbkp * EBLK - e
    dump = (n + (jnp.arange(npads, dtype=jnp.int32) % 16)).astype(jnp.int32)
    row2d, col2d = _pad_edges(edge_index[0], edge_index[1], dump, dump, nbkp)

    npads_i = idx_rows_i * EBLK - e_ind
    zpad_i = jnp.zeros((npads_i,), jnp.int32)
    i0_2d, i1_2d = _pad_edges(ind_edge[0], ind_edge[1], zpad_i, zpad_i,
                              idx_rows_i)

    zeros_nd = jnp.zeros((npad, 128), f32)

    ls = params["layers"]

    def wpack(layer):
        b3 = jnp.zeros((8, 128), f32)
        b3 = b3.at[0].set(layer["b_sd"]).at[1].set(layer["b_ds"])
        b3 = b3.at[2].set(layer["b_fc"])
        b3 = b3.at[3, 0].set(layer["b_outf"][0]).at[3, 1].set(layer["b_inf"][0])
        to = _ceil_to(layer["out_tab"].shape[0], 8)
        ti = _ceil_to(layer["in_tab"].shape[0], 8)
        otab = jnp.pad(layer["out_tab"],
                       ((0, to - layer["out_tab"].shape[0]), (0, 0)))
        itab = jnp.pad(layer["in_tab"],
                       ((0, ti - layer["in_tab"].shape[0]), (0, 0)))
        return (otab, itab, layer["W_outf"], layer["W_inf"], layer["W_sd"],
                layer["W_ds"], layer["W_fc"], b3)

    w0 = wpack(ls[0])
    w1p = wpack(ls[1])
    wl1 = params["W_lin"][:128]
    wl2 = params["W_lin"][128:]
    blin = jnp.zeros((8, 40), f32).at[0].set(params["b_lin"])

    conv = _make_conv_kernel(npad, nbkp)
    pair = _make_pair_kernel(npad, nbk_i, idx_rows_i)

    xs0, xo0, inv2 = _prep(xp, deg2)
    part_out0, part_in0 = conv(xs0, xo0, row2d, col2d, zeros_nd)
    x1, xs1, xo1, cp0 = _dense_layer(0, xp, part_out0, part_in0, inv2, deg2,
                                     m4, *w0)
    part_out1, part_in1 = conv(xs1, xo1, row2d, col2d, zeros_nd)
    p1, p2, cpf = _dense_layer(1, x1, part_out1, part_in1, inv2, deg2, m4,
                               *w1p, cp0=cp0, w1=wl1, w2=wl2, blin=blin)
    o4 = pair(p1, p2, i0_2d, i1_2d)
    logits = o4.transpose(1, 3, 0, 2).reshape(e_ind, 40)
    c_ins = cpf[:n, 1:2]
    c_outs = cpf[:n, 0:1]
    return logits, c_ins, c_outs


# speedup vs baseline: 1.8306x; 1.3191x over previous
import functools

import jax
import jax.numpy as jnp
from jax import lax
from jax.experimental import pallas as pl
from jax.experimental.pallas import tpu as pltpu
from jax.experimental.pallas import tpu_sc as plsc

ALPHA_C = 0.5
NBLK = 1024
EBLK = 128
NW = 32
LSM_BLK = 4000


def _ceil_to(x, m):
    return (x + m - 1) // m * m


def _prep_body(x_ref, deg_ref, xs_ref, xo_ref, inv_ref):
    deg = deg_ref[...].astype(jnp.float32)
    inv = jnp.where(deg > 0, lax.rsqrt(jnp.maximum(deg, 1.0)), 0.0)
    x = x_ref[...]
    xo_ref[...] = inv[:, 0:1] * x
    xs_ref[...] = inv[:, 1:2] * x
    inv_ref[...] = inv


def _prep(xp, deg2):
    npad = xp.shape[0]
    grid = (npad // NBLK,)
    return pl.pallas_call(
        _prep_body,
        grid=grid,
        in_specs=[
            pl.BlockSpec((NBLK, 128), lambda i: (i, 0)),
            pl.BlockSpec((NBLK, 2), lambda i: (i, 0)),
        ],
        out_specs=[
            pl.BlockSpec((NBLK, 128), lambda i: (i, 0)),
            pl.BlockSpec((NBLK, 128), lambda i: (i, 0)),
            pl.BlockSpec((NBLK, 2), lambda i: (i, 0)),
        ],
        out_shape=[
            jax.ShapeDtypeStruct((npad, 128), jnp.float32),
            jax.ShapeDtypeStruct((npad, 128), jnp.float32),
            jax.ShapeDtypeStruct((npad, 2), jnp.float32),
        ],
    )(xp, deg2)


def _layer_out(x, sout, sin, inv, deg, m4, otab, itab, wof, wif,
               wsd, wds, wfc, b3):
    out_inv = inv[:, 0:1]
    in_inv = inv[:, 1:2]
    out_nei = out_inv * sout
    in_nei = in_inv * sin
    nb = x.shape[0]
    to = otab.shape[0]
    ti = itab.shape[0]
    oh_o = (lax.broadcasted_iota(jnp.int32, (nb, to), 1) == deg[:, 0:1]
            ).astype(jnp.float32)
    oh_i = (lax.broadcasted_iota(jnp.int32, (nb, ti), 1) == deg[:, 1:2]
            ).astype(jnp.float32)
    tgo = jnp.dot(oh_o, otab, preferred_element_type=jnp.float32)
    tgi = jnp.dot(oh_i, itab, preferred_element_type=jnp.float32)
    co = jnp.dot(out_nei - x + tgo, wof, preferred_element_type=jnp.float32)
    co = co + b3[3:4, 0:1]
    ci = jnp.dot(in_nei - x + tgi, wif, preferred_element_type=jnp.float32)
    ci = ci + b3[3:4, 1:2]
    c0 = 1.0 / (1.0 + jnp.exp(ci - co))
    c1 = 1.0 - c0
    c_out = c0 * m4[:, 0:1] + m4[:, 1:2]
    c_in = c1 * m4[:, 2:3] + m4[:, 3:4]
    h = (c_out * (jnp.dot(out_nei, wsd, preferred_element_type=jnp.float32)
                  + b3[0:1, :])
         + c_in * (jnp.dot(in_nei, wds, preferred_element_type=jnp.float32)
                   + b3[1:2, :])
         + ALPHA_C * (jnp.dot(x, wfc, preferred_element_type=jnp.float32)
                      + b3[2:3, :]))
    xn = jnp.maximum(h, 0.0)
    return xn, c_out, c_in


def _b0_body(x_ref, po0, po1, pi0, pi1, inv_ref, deg_ref, m4_ref,
             otab_ref, itab_ref, wof_ref, wif_ref, wsd_ref, wds_ref,
             wfc_ref, b3_ref, x1_ref, xs1_ref, xo1_ref, cp_ref):
    sout = po0[0] + po1[0]
    sin = pi0[0] + pi1[0]
    xn, c_out, c_in = _layer_out(
        x_ref[...], sout, sin, inv_ref[...], deg_ref[...], m4_ref[...],
        otab_ref[...], itab_ref[...], wof_ref[...], wif_ref[...],
        wsd_ref[...], wds_ref[...], wfc_ref[...], b3_ref[...])
    x1_ref[...] = xn
    inv = inv_ref[...]
    xo1_ref[...] = inv[:, 0:1] * xn
    xs1_ref[...] = inv[:, 1:2] * xn
    cp_ref[...] = jnp.concatenate([c_out, c_in], axis=1)


def _b1_body(x_ref, po0, po1, pi0, pi1, inv_ref, deg_ref, m4_ref,
             otab_ref, itab_ref, wof_ref, wif_ref, wsd_ref, wds_ref,
             wfc_ref, b3_ref, cp0_ref, w1_ref, w2_ref, blin_ref,
             p1_ref, p2_ref, cpf_ref):
    sout = po0[0] + po1[0]
    sin = pi0[0] + pi1[0]
    x1 = x_ref[...]
    xn, c_out, c_in = _layer_out(
        x1, sout, sin, inv_ref[...], deg_ref[...], m4_ref[...],
        otab_ref[...], itab_ref[...], wof_ref[...], wif_ref[...],
        wsd_ref[...], wds_ref[...], wfc_ref[...], b3_ref[...])
    xj = jnp.maximum(x1, xn)
    nb = xj.shape[0]
    p1 = jnp.dot(xj, w1_ref[...], preferred_element_type=jnp.float32)
    p1 = p1 + blin_ref[0:1, :]
    p2 = jnp.dot(xj, w2_ref[...], preferred_element_type=jnp.float32)
    p1_ref[...] = jnp.concatenate(
        [p1, jnp.full((nb, 8), -1e30, dtype=jnp.float32)], axis=1)
    p2_ref[...] = jnp.concatenate(
        [p2, jnp.zeros((nb, 8), dtype=jnp.float32)], axis=1)
    cpf_ref[...] = (cp0_ref[...] + jnp.concatenate([c_out, c_in], axis=1)) * 0.5


def _dense_layer(lidx, xp, part_out, part_in, inv2, deg2, m4,
                 otab, itab, wof, wif, wsd, wds, wfc, b3,
                 cp0=None, w1=None, w2=None, blin=None):
    npad = xp.shape[0]
    grid = (npad // NBLK,)
    to = otab.shape[0]
    ti = itab.shape[0]
    node_spec = lambda w: pl.BlockSpec((NBLK, w), lambda i: (i, 0))
    part_spec = lambda c: pl.BlockSpec((1, NBLK, 128), lambda i, c=c: (c, i, 0))
    full = lambda shape: pl.BlockSpec(shape, lambda i, s=shape: tuple(
        0 for _ in s))
    in_specs = [
        node_spec(128),
        part_spec(0), part_spec(1),
        part_spec(0), part_spec(1),
        node_spec(2), node_spec(2), node_spec(4),
        full((to, 128)), full((ti, 128)),
        full((128, 1)), full((128, 1)),
        full((128, 128)), full((128, 128)), full((128, 128)),
        full((8, 128)),
    ]
    args = [xp, part_out, part_out, part_in, part_in, inv2, deg2, m4,
            otab, itab, wof, wif, wsd, wds, wfc, b3]
    if lidx == 0:
        out_specs = [node_spec(128), node_spec(128), node_spec(128),
                     node_spec(2)]
        out_shape = [jax.ShapeDtypeStruct((npad, 128), jnp.float32)] * 3 + [
            jax.ShapeDtypeStruct((npad, 2), jnp.float32)]
        body = _b0_body
    else:
        in_specs += [node_spec(2), full((128, 40)), full((128, 40)),
                     full((8, 40))]
        args += [cp0, w1, w2, blin]
        out_specs = [node_spec(48), node_spec(48), node_spec(2)]
        out_shape = [jax.ShapeDtypeStruct((npad, 48), jnp.float32)] * 2 + [
            jax.ShapeDtypeStruct((npad, 2), jnp.float32)]
        body = _b1_body
    return pl.pallas_call(
        body, grid=grid, in_specs=in_specs, out_specs=out_specs,
        out_shape=out_shape)(*args)


def _lsm_body(z_ref, o_ref):
    z = z_ref[...]
    m = jnp.max(z, axis=1, keepdims=True)
    e = jnp.exp(z - m)
    s = jnp.sum(e, axis=1, keepdims=True)
    o_ref[...] = z[:, :40] - (m + jnp.log(s))


def _log_softmax(z, e_out):
    grid = (e_out // LSM_BLK,)
    return pl.pallas_call(
        _lsm_body,
        grid=grid,
        in_specs=[pl.BlockSpec((LSM_BLK, 48), lambda i: (i, 0))],
        out_specs=pl.BlockSpec((LSM_BLK, 40), lambda i: (i, 0)),
        out_shape=jax.ShapeDtypeStruct((e_out, 40), jnp.float32),
    )(z)


_NBUF = 2


@functools.lru_cache(maxsize=None)
def _make_conv_kernel(npad, nbkp):
    pw = nbkp // NW
    hw = pw // 2
    assert pw % 2 == 0 and hw % 8 == 0 and pw % _NBUF == 0
    rows_z = npad // 16
    mesh = plsc.VectorSubcoreMesh(core_axis_name="c", subcore_axis_name="s")

    @functools.partial(
        pl.kernel,
        mesh=mesh,
        name="conv_segsum",
        out_type=(
            jax.ShapeDtypeStruct((2, npad, 128), jnp.float32),
            jax.ShapeDtypeStruct((2, npad, 128), jnp.float32),
        ),
        scratch_types=(
            [pltpu.VMEM((pw, EBLK), jnp.int32),
             pltpu.VMEM((hw, EBLK), jnp.int32)]
            + [pltpu.VMEM((EBLK, 128), jnp.float32) for _ in range(_NBUF)]
            + [pltpu.VMEM_SHARED((npad, 128), jnp.float32)]
            + [pltpu.SemaphoreType.DMA for _ in range(_NBUF)]
        ),
    )
    def conv(xs_hbm, xo_hbm, row2d, col2d, zeros_hbm, sout_hbm, sin_hbm,
             gbuf, sbuf, d0, d1, acc, s0, s1):
        dbufs = (d0, d1)
        sems = (s0, s1)
        c = lax.axis_index("c")
        s = lax.axis_index("s")
        w = s * 2 + c
        start = w * pw

        def phase(table_hbm, g2d, s2d, out_prev, out_hbm):
            pltpu.sync_copy(g2d.at[pl.ds(start, pw)], gbuf)
            pltpu.sync_copy(s2d.at[pl.ds(start, hw)], sbuf)
            for b in range(_NBUF):
                pltpu.async_copy(table_hbm.at[gbuf.at[b]], dbufs[b], sems[b])
            if out_prev is not None:
                pltpu.sync_copy(acc.at[pl.ds(s * rows_z, rows_z)],
                                out_prev.at[c, pl.ds(s * rows_z, rows_z)])
            pltpu.sync_copy(zeros_hbm.at[pl.ds(s * rows_z, rows_z)],
                            acc.at[pl.ds(s * rows_z, rows_z)])
            plsc.subcore_barrier()

            def body(i, carry):
                for b in range(_NBUF):
                    j = i * _NBUF + b

                    @pl.when(j == hw)
                    def _():
                        pltpu.sync_copy(s2d.at[pl.ds(start + hw, hw)], sbuf)

                    jl = lax.select(j >= hw, j - hw, j)
                    pltpu.make_async_copy(table_hbm.at[gbuf.at[j]], dbufs[b],
                                          sems[b]).wait()
                    pltpu.sync_copy(dbufs[b], acc.at[sbuf.at[jl]], add=True)
                    jn = j + _NBUF

                    @pl.when(jn < pw)
                    def _():
                        pltpu.async_copy(table_hbm.at[gbuf.at[jn]], dbufs[b],
                                         sems[b])
                return carry

            lax.fori_loop(0, pw // _NBUF, body, 0)
            plsc.subcore_barrier()

        phase(xs_hbm, col2d, row2d, None, sout_hbm)
        phase(xo_hbm, row2d, col2d, sout_hbm, sin_hbm)
        pltpu.sync_copy(acc.at[pl.ds(s * rows_z, rows_z)],
                        sin_hbm.at[c, pl.ds(s * rows_z, rows_z)])

    return conv


_PNBUF = 2


@functools.lru_cache(maxsize=None)
def _make_pair_kernel(npad, nbk, idx_rows):
    base = nbk // NW
    rem = nbk % NW
    pwm = base + (1 if rem else 0)
    niter = _ceil_to(pwm, _PNBUF) // _PNBUF
    mesh = plsc.VectorSubcoreMesh(core_axis_name="c", subcore_axis_name="s")

    @functools.partial(
        pl.kernel,
        mesh=mesh,
        name="pair_head",
        out_type=jax.ShapeDtypeStruct((5, nbk, 8, EBLK), jnp.float32),
        compiler_params=pltpu.CompilerParams(use_tc_tiling_on_sc=False,
                                             needs_layout_passes=False),
        scratch_types=(
            [pltpu.VMEM((pwm, EBLK), jnp.int32),
             pltpu.VMEM((pwm, EBLK), jnp.int32)]
            + [pltpu.VMEM((EBLK, 48), jnp.float32) for _ in range(2 * _PNBUF)]
            + [pltpu.VMEM((EBLK, 16), jnp.float32),
               pltpu.VMEM((EBLK, 16), jnp.float32),
               pltpu.VMEM((40, EBLK), jnp.float32)]
            + [pltpu.SemaphoreType.DMA for _ in range(2 * _PNBUF)]
        ),
    )
    def pair(p1_hbm, p2_hbm, i0_2d, i1_2d, o4_hbm, b0, b1,
             ga0, ga1, gb0, gb1, mbuf, sbuf, obuf, sa0, sa1, sb0, sb1):
        g1s = (ga0, ga1)
        g2s = (gb0, gb1)
        sem1s = (sa0, sa1)
        sem2s = (sb0, sb1)
        c = lax.axis_index("c")
        s = lax.axis_index("s")
        w = s * 2 + c
        start = w * base + jnp.minimum(w, rem)
        cnt = base + (w < rem).astype(jnp.int32)
        pltpu.sync_copy(i0_2d.at[pl.ds(start, pwm)], b0)
        pltpu.sync_copy(i1_2d.at[pl.ds(start, pwm)], b1)
        for b in range(_PNBUF):
            pltpu.async_copy(p1_hbm.at[b0.at[b]], g1s[b], sem1s[b])
            pltpu.async_copy(p2_hbm.at[b1.at[b]], g2s[b], sem2s[b])

        def body(i, carry):
            for b in range(_PNBUF):
                j = i * _PNBUF + b
                g1, g2 = g1s[b], g2s[b]

                @pl.when(j < cnt)
                def _blk():
                    pltpu.make_async_copy(p1_hbm.at[b0.at[j]], g1,
                                          sem1s[b]).wait()
                    pltpu.make_async_copy(p2_hbm.at[b1.at[j]], g2,
                                          sem2s[b]).wait()

                    @plsc.parallel_loop(0, EBLK, unroll=4)
                    def _row(r):
                        a0 = g1[r, pl.ds(0, 16)] + g2[r, pl.ds(0, 16)]
                        a1 = g1[r, pl.ds(16, 16)] + g2[r, pl.ds(16, 16)]
                        a2 = g1[r, pl.ds(32, 16)] + g2[r, pl.ds(32, 16)]
                        g1[r, pl.ds(0, 16)] = a0
                        g1[r, pl.ds(16, 16)] = a1
                        g1[r, pl.ds(32, 16)] = a2
                        m = jnp.max(jnp.maximum(jnp.maximum(a0, a1), a2))
                        mb = jnp.full((16,), m, dtype=jnp.float32)
                        e = (jnp.exp(a0 - mb) + jnp.exp(a1 - mb)
                             + jnp.exp(a2 - mb))
                        s = jnp.sum(e)
                        mbuf[r, :] = mb
                        sbuf[r, :] = jnp.full((16,), s, dtype=jnp.float32)

                    @plsc.parallel_loop(0, 8, unroll=2)
                    def _grp(g):
                        lane = jax.lax.broadcasted_iota(jnp.int32, (16,), 0)
                        rows16 = g * 16 + lane
                        mv = plsc.load_gather(mbuf, [rows16, lane])
                        sv = plsc.load_gather(sbuf, [rows16, lane])
                        bits = plsc.bitcast(sv, jnp.int32)
                        y = ((bits - 1065353216).astype(jnp.float32)
                             * 8.2629583e-08)
                        for _ in range(3):
                            y = y - 1.0 + sv * jnp.exp(-y)
                        l = mv + y
                        for jj in range(40):
                            zj = plsc.load_gather(
                                g1, [rows16, jnp.full((16,), jj, jnp.int32)])
                            obuf[jj, pl.ds(g * 16, 16)] = zj - l

                    blk = start + j
                    for jt in range(5):
                        pltpu.sync_copy(obuf.at[pl.ds(jt * 8, 8)],
                                        o4_hbm.at[jt, blk])
                    jn = j + _PNBUF

                    @pl.when(jn < cnt)
                    def _():
                        pltpu.async_copy(p1_hbm.at[b0.at[jn]], g1, sem1s[b])
                        pltpu.async_copy(p2_hbm.at[b1.at[jn]], g2, sem2s[b])
            return carry

        lax.fori_loop(0, niter, body, 0)

    return pair


def _pad_edges(idx_a, idx_b, pad_a, pad_b, nbkp):
    ep = nbkp * EBLK
    e = idx_a.shape[0]
    a = jnp.concatenate([idx_a, pad_a[: ep - e]])
    b = jnp.concatenate([idx_b, pad_b[: ep - e]])
    return a.reshape(nbkp, EBLK), b.reshape(nbkp, EBLK)


def kernel(x, edge_index, ind_edge, in_degree, out_degree, masks, params):
    n, d = x.shape
    e = edge_index.shape[1]
    e_ind = ind_edge.shape[1]
    npad = _ceil_to(n + 16, NBLK)
    nbkp = _ceil_to(-(-e // EBLK), NW * 8)
    nbk_i = -(-e_ind // EBLK)
    assert e_ind % EBLK == 0
    pwm_i = nbk_i // NW + (1 if nbk_i % NW else 0)
    idx_rows_i = _ceil_to((NW - 1) * (nbk_i // NW) + (nbk_i % NW) + pwm_i, 8)

    f32 = jnp.float32
    pad_n = npad - n
    xp = jnp.pad(x, ((0, pad_n), (0, 0)))
    deg2 = jnp.pad(
        jnp.stack([out_degree, in_degree], axis=1), ((0, pad_n), (0, 0)))
    m4 = jnp.pad(
        jnp.stack([masks["out_deg_mask"], masks["out_deg_mask_bias"],
                   masks["in_deg_mask"], masks["in_deg_mask_bias"]], axis=1),
        ((0, pad_n), (0, 0)))

    npads = nbkp * EBLK - e
    dump = (n + (jnp.arange(npads, dtype=jnp.int32) % 16)).astype(jnp.int32)
    row2d, col2d = _pad_edges(edge_index[0], edge_index[1], dump, dump, nbkp)

    npads_i = idx_rows_i * EBLK - e_ind
    zpad_i = jnp.zeros((npads_i,), jnp.int32)
    i0_2d, i1_2d = _pad_edges(ind_edge[0], ind_edge[1], zpad_i, zpad_i,
                              idx_rows_i)

    zeros_nd = jnp.zeros((npad, 128), f32)

    ls = params["layers"]

    def wpack(layer):
        b3 = jnp.zeros((8, 128), f32)
        b3 = b3.at[0].set(layer["b_sd"]).at[1].set(layer["b_ds"])
        b3 = b3.at[2].set(layer["b_fc"])
        b3 = b3.at[3, 0].set(layer["b_outf"][0]).at[3, 1].set(layer["b_inf"][0])
        to = _ceil_to(layer["out_tab"].shape[0], 8)
        ti = _ceil_to(layer["in_tab"].shape[0], 8)
        otab = jnp.pad(layer["out_tab"],
                       ((0, to - layer["out_tab"].shape[0]), (0, 0)))
        itab = jnp.pad(layer["in_tab"],
                       ((0, ti - layer["in_tab"].shape[0]), (0, 0)))
        return (otab, itab, layer["W_outf"], layer["W_inf"], layer["W_sd"],
                layer["W_ds"], layer["W_fc"], b3)

    w0 = wpack(ls[0])
    w1p = wpack(ls[1])
    wl1 = params["W_lin"][:128]
    wl2 = params["W_lin"][128:]
    blin = jnp.zeros((8, 40), f32).at[0].set(params["b_lin"])

    conv = _make_conv_kernel(npad, nbkp)
    pair = _make_pair_kernel(npad, nbk_i, idx_rows_i)

    xs0, xo0, inv2 = _prep(xp, deg2)
    part_out0, part_in0 = conv(xs0, xo0, row2d, col2d, zeros_nd)
    x1, xs1, xo1, cp0 = _dense_layer(0, xp, part_out0, part_in0, inv2, deg2,
                                     m4, *w0)
    part_out1, part_in1 = conv(xs1, xo1, row2d, col2d, zeros_nd)
    p1, p2, cpf = _dense_layer(1, x1, part_out1, part_in1, inv2, deg2, m4,
                               *w1p, cp0=cp0, w1=wl1, w2=wl2, blin=blin)
    o4 = pair(p1, p2, i0_2d, i1_2d)
    logits = o4.transpose(1, 3, 0, 2).reshape(e_ind, 40)
    c_ins = cpf[:n, 1:2]
    c_outs = cpf[:n, 0:1]
    return logits, c_ins, c_outs
